# baseline (speedup 1.0000x reference)
import functools

import jax
import jax.numpy as jnp
from jax import lax
from jax.experimental import pallas as pl
from jax.experimental.pallas import tpu as pltpu

NEG_SLOPE = 0.01
TM = 128
F32 = jnp.float32
VMEM_LIMIT = 56 * 1024 * 1024


def _lrelu(x):
    return jnp.where(x > 0, x, NEG_SLOPE * x)


def _round_up(n, m):
    return ((n + m - 1) // m) * m


def _pad_rows(x, rows):
    return jnp.pad(x.astype(F32), ((0, rows - x.shape[0]), (0, 0)))


def _pad_idx(idx, rows, fill):
    return jnp.pad(idx.astype(jnp.int32), (0, rows - idx.shape[0]),
                   constant_values=fill)


def _rep(shape):
    return pl.BlockSpec(shape, lambda *_: (0,) * len(shape))


def _r2(b):
    return b.reshape(1, -1)


def _ff_proj_kernel(xn_ref, xg_ref,
                    wn1, bn1, wn2, bn2,
                    wg1, bg1, wg2, bg2,
                    wps, wpdv, wpdu,
                    vu_ref, pp_ref):
    xn = xn_ref[...]
    hv = _lrelu(jnp.dot(xn, wn1[...], preferred_element_type=F32) + bn1[...])
    v = _lrelu(jnp.dot(hv, wn2[...], preferred_element_type=F32) + bn2[...])
    xg = xg_ref[...]
    hu = _lrelu(jnp.dot(xg, wg1[...], preferred_element_type=F32) + bg1[...])
    u = _lrelu(jnp.dot(hu, wg2[...], preferred_element_type=F32) + bg2[...])
    vu_ref[...] = jnp.concatenate([v, u], axis=1)
    ps = jnp.dot(v, wps[...], preferred_element_type=F32)
    pd = (jnp.dot(v, wpdv[...], preferred_element_type=F32)
          + jnp.dot(u, wpdu[...], preferred_element_type=F32))
    pp_ref[...] = jnp.concatenate([ps, pd], axis=1)


def _edge_kernel(tiles_per_core, n_edges,
                 xe_ref, src_ref, dst_ref, dstp_ref, pp_ref,
                 we1, be1, we2, be2, wee, b1, w2, b2,
                 eout_ref, eagg_ref, gs_scr, gd_scr, pay_scr):
    i = pl.program_id(1)

    @pl.when(i == 0)
    def _():
        eagg_ref[...] = jnp.zeros_like(eagg_ref)
        pay_scr[...] = jnp.zeros(pay_scr.shape, pay_scr.dtype)

    U = 4
    for b in range(TM // U):
        ds = [dstp_ref[0, 0, b * U + j] for j in range(U)]
        pays = [pay_scr[b * U + j] for j in range(U)]
        adj = []
        for j in range(U):
            pj = pays[j]
            for k in range(j):
                pj = pj + (ds[j] == ds[k]).astype(F32) * pays[k]
            adj.append(pj)
        loads = [eagg_ref[ds[j], 0] for j in range(U)]
        for j in range(U):
            eagg_ref[ds[j], 0] = loads[j] + adj[j]

    xe = xe_ref[...]
    h = _lrelu(jnp.dot(xe, we1[...], preferred_element_type=F32) + be1[...])
    e = _lrelu(jnp.dot(h, we2[...], preferred_element_type=F32) + be2[...])
    q = jnp.dot(e, wee[...], preferred_element_type=F32) + b1[...]

    for mi in range(TM):
        gs_scr[mi] = pp_ref[src_ref[0, 0, mi], 0]
        gd_scr[mi] = pp_ref[dst_ref[0, 0, mi], 0]

    hh = gs_scr.shape[1] // 2
    h1 = _lrelu(gs_scr[:, :hh] + gd_scr[:, hh:] + q)
    e_new = _lrelu(jnp.dot(h1, w2[...], preferred_element_type=F32) + b2[...])
    eout_ref[...] = e_new + xe

    ii = jnp.minimum(i, tiles_per_core - 1)
    base = (pl.program_id(0) * tiles_per_core + ii) * TM
    rows = lax.broadcasted_iota(jnp.int32, (TM, 1), 0) + base
    valid = (rows < n_edges).astype(F32)
    pay_scr[...] = jnp.concatenate([e_new, jnp.ones_like(e_new)], axis=1) * valid


def _node_kernel(vu_ref, ega_ref, egb_ref, xn_ref, ng_ref,
                 w_vu, w_ef, b1, w2, b2,
                 nout_ref, pool_ref):
    j = pl.program_id(1)

    @pl.when(j == 0)
    def _():
        pool_ref[...] = jnp.zeros_like(pool_ref)

    o = nout_ref.shape[1]
    agg = ega_ref[...] + egb_ref[...]
    ef_sum = agg[:, :o]
    deg = agg[:, o:]
    ef = ef_sum * pl.reciprocal(jnp.maximum(deg, 1.0), approx=True)
    vu = vu_ref[...]
    h = _lrelu(jnp.dot(vu, w_vu[...], preferred_element_type=F32)
               + jnp.dot(ef, w_ef[...], preferred_element_type=F32)
               + b1[...])
    n_new = _lrelu(jnp.dot(h, w2[...], preferred_element_type=F32) + b2[...])
    nout_ref[...] = n_new + xn_ref[...]

    gp = pool_ref.shape[0]
    u = vu[:, vu.shape[1] // 2:]
    pooled = jnp.concatenate([n_new, ef_sum, u], axis=1)
    row_ids = lax.broadcasted_iota(jnp.int32, (gp, TM), 0)
    oh = (row_ids == ng_ref[...]).astype(F32)
    pool_ref[...] += jnp.dot(oh, pooled, preferred_element_type=F32)


def _graph_kernel(pa_ref, pb_ref, cntn_ref, cnte_ref, ng_ref, xg_ref,
                  w1, b1, w2, b2,
                  gout_ref, gnew_scr):
    j = pl.program_id(1)
    o = gout_ref.shape[1]

    @pl.when(j == 0)
    def _():
        pool = pa_ref[...] + pb_ref[...]
        gp, width = pool.shape
        inv_n = pl.reciprocal(jnp.maximum(cntn_ref[...], 1.0), approx=True)
        inv_e = pl.reciprocal(jnp.maximum(cnte_ref[...], 1.0), approx=True)
        lane = lax.broadcasted_iota(jnp.int32, (gp, width), 1)
        scale = jnp.where(lane < o, inv_n, jnp.where(lane < 2 * o, inv_e, inv_n))
        cat_g = pool * scale
        hg = _lrelu(jnp.dot(cat_g, w1[...], preferred_element_type=F32) + b1[...])
        gnew_scr[...] = _lrelu(jnp.dot(hg, w2[...], preferred_element_type=F32)
                               + b2[...])

    gp = gnew_scr.shape[0]
    col_ids = lax.broadcasted_iota(jnp.int32, (TM, gp), 1)
    oh = (col_ids == ng_ref[...]).astype(F32)
    gout_ref[...] = (jnp.dot(oh, gnew_scr[...], preferred_element_type=F32)
                     + xg_ref[...])


def kernel(ff_node_w1, ff_node_b1, ff_node_w2, ff_node_b2,
           ff_edge_w1, ff_edge_b1, ff_edge_w2, ff_edge_b2,
           ff_graph_w1, ff_graph_b1, ff_graph_w2, ff_graph_b2,
           edge_update_w1, edge_update_b1, edge_update_w2, edge_update_b2,
           node_update_w1, node_update_b1, node_update_w2, node_update_b2,
           graph_update_w1, graph_update_b1, graph_update_w2, graph_update_b2,
           node_feats, edge_feats, graph_feats,
           src, dst, node_graph, batch_num_nodes, batch_num_edges):
    N, D = node_feats.shape
    E = edge_feats.shape[0]
    G = batch_num_nodes.shape[0]
    H = ff_node_w2.shape[1]
    O = edge_update_w2.shape[1]

    Np = _round_up(max(N, 1), 2 * TM)
    Ep = _round_up(max(E, 1), 2 * TM)
    Gp = _round_up(max(G, 1), 8)
    gN = Np // TM
    gN2 = gN // 2
    gE2 = Ep // TM // 2

    w_ps = edge_update_w1[0:H]
    w_pdv = edge_update_w1[H:2 * H]
    w_ee = edge_update_w1[2 * H:3 * H]
    w_pdu = edge_update_w1[3 * H:4 * H]
    w_vu = jnp.concatenate([node_update_w1[0:H],
                            node_update_w1[H + O:H + O + H]], axis=0)
    w_ef = node_update_w1[H:H + O]

    xn = _pad_rows(node_feats, Np)
    xg = _pad_rows(graph_feats, Np)
    xe = _pad_rows(edge_feats, Ep)
    src_t = _pad_idx(src, Ep, 0).reshape(Ep // TM, 1, TM)
    dst_t = _pad_idx(dst, Ep, 0).reshape(Ep // TM, 1, TM)
    ng_r = _pad_idx(node_graph, Np, -1).reshape(1, Np)
    ng_c = _pad_idx(node_graph, Np, -1).reshape(Np, 1)
    cnt_n = jnp.zeros((Gp, 1), F32).at[:G, 0].set(batch_num_nodes.astype(F32))
    cnt_e = jnp.zeros((Gp, 1), F32).at[:G, 0].set(batch_num_edges.astype(F32))

    cp_par = pltpu.CompilerParams(dimension_semantics=("parallel",),
                                  vmem_limit_bytes=VMEM_LIMIT)
    cp_mix = pltpu.CompilerParams(dimension_semantics=("parallel", "arbitrary"),
                                  vmem_limit_bytes=VMEM_LIMIT)

    vu, pp = pl.pallas_call(
        _ff_proj_kernel,
        out_shape=(jax.ShapeDtypeStruct((Np, 2 * H), F32),
                   jax.ShapeDtypeStruct((Np, 2 * H), F32)),
        grid=(gN,),
        in_specs=[
            pl.BlockSpec((TM, D), lambda i: (i, 0)),
            pl.BlockSpec((TM, D), lambda i: (i, 0)),
            _rep((D, D)), _rep((1, D)), _rep((D, H)), _rep((1, H)),
            _rep((D, D)), _rep((1, D)), _rep((D, H)), _rep((1, H)),
            _rep((H, H)), _rep((H, H)), _rep((H, H)),
        ],
        out_specs=(pl.BlockSpec((TM, 2 * H), lambda i: (i, 0)),
                   pl.BlockSpec((TM, 2 * H), lambda i: (i, 0))),
        compiler_params=cp_par,
    )(xn, xg,
      ff_node_w1, _r2(ff_node_b1), ff_node_w2, _r2(ff_node_b2),
      ff_graph_w1, _r2(ff_graph_b1), ff_graph_w2, _r2(ff_graph_b2),
      w_ps, w_pdv, w_pdu)

    pp3 = pp.reshape(Np, 1, 2 * H)
    eout, eagg = pl.pallas_call(
        functools.partial(_edge_kernel, gE2, E),
        out_shape=(jax.ShapeDtypeStruct((Ep, O), F32),
                   jax.ShapeDtypeStruct((2 * Np, 1, 2 * O), F32)),
        grid=(2, gE2 + 1),
        in_specs=[
            pl.BlockSpec((TM, D),
                         lambda c, i, g=gE2: (c * g + jnp.minimum(i, g - 1), 0)),
            pl.BlockSpec((1, 1, TM),
                         lambda c, i, g=gE2: (c * g + jnp.minimum(i, g - 1), 0, 0),
                         memory_space=pltpu.SMEM),
            pl.BlockSpec((1, 1, TM),
                         lambda c, i, g=gE2: (c * g + jnp.minimum(i, g - 1), 0, 0),
                         memory_space=pltpu.SMEM),
            pl.BlockSpec((1, 1, TM),
                         lambda c, i, g=gE2: (c * g + jnp.maximum(i - 1, 0), 0, 0),
                         memory_space=pltpu.SMEM),
            pl.BlockSpec((Np, 1, 2 * H), lambda c, i: (0, 0, 0)),
            _rep((D, D)), _rep((1, D)), _rep((D, H)), _rep((1, H)),
            _rep((H, H)), _rep((1, H)), _rep((H, O)), _rep((1, O)),
        ],
        out_specs=(pl.BlockSpec((TM, O),
                                lambda c, i, g=gE2: (c * g + jnp.minimum(i, g - 1), 0)),
                   pl.BlockSpec((Np, 1, 2 * O), lambda c, i: (c, 0, 0))),
        scratch_shapes=[pltpu.VMEM((TM, 2 * H), F32),
                        pltpu.VMEM((TM, 2 * H), F32),
                        pltpu.VMEM((TM, 2 * O), F32)],
        compiler_params=cp_mix,
    )(xe, src_t, dst_t, dst_t, pp3,
      ff_edge_w1, _r2(ff_edge_b1), ff_edge_w2, _r2(ff_edge_b2),
      w_ee, _r2(edge_update_b1), edge_update_w2, _r2(edge_update_b2))

    eagg2 = eagg.reshape(2 * Np, 2 * O)
    nout, pool = pl.pallas_call(
        _node_kernel,
        out_shape=(jax.ShapeDtypeStruct((Np, O), F32),
                   jax.ShapeDtypeStruct((2 * Gp, 2 * O + H), F32)),
        grid=(2, gN2),
        in_specs=[
            pl.BlockSpec((TM, 2 * H), lambda c, j, g=gN2: (c * g + j, 0)),
            pl.BlockSpec((TM, 2 * O), lambda c, j, g=gN2: (c * g + j, 0)),
            pl.BlockSpec((TM, 2 * O), lambda c, j, g=gN2, n=gN: (c * g + j + n, 0)),
            pl.BlockSpec((TM, D), lambda c, j, g=gN2: (c * g + j, 0)),
            pl.BlockSpec((1, TM), lambda c, j, g=gN2: (0, c * g + j)),
            _rep((2 * H, H)), _rep((O, H)), _rep((1, H)),
            _rep((H, O)), _rep((1, O)),
        ],
        out_specs=(pl.BlockSpec((TM, O), lambda c, j, g=gN2: (c * g + j, 0)),
                   pl.BlockSpec((Gp, 2 * O + H), lambda c, j: (c, 0))),
        compiler_params=cp_mix,
    )(vu, eagg2, eagg2, xn, ng_r,
      w_vu, w_ef, _r2(node_update_b1), node_update_w2, _r2(node_update_b2))

    gout = pl.pallas_call(
        _graph_kernel,
        out_shape=jax.ShapeDtypeStruct((Np, O), F32),
        grid=(2, gN2),
        in_specs=[
            pl.BlockSpec((Gp, 2 * O + H), lambda c, j: (0, 0)),
            pl.BlockSpec((Gp, 2 * O + H), lambda c, j: (1, 0)),
            _rep((Gp, 1)), _rep((Gp, 1)),
            pl.BlockSpec((TM, 1), lambda c, j, g=gN2: (c * g + j, 0)),
            pl.BlockSpec((TM, D), lambda c, j, g=gN2: (c * g + j, 0)),
            _rep((2 * O + H, H)), _rep((1, H)), _rep((H, O)), _rep((1, O)),
        ],
        out_specs=pl.BlockSpec((TM, O), lambda c, j, g=gN2: (c * g + j, 0)),
        scratch_shapes=[pltpu.VMEM((Gp, O), F32)],
        compiler_params=cp_mix,
    )(pool, pool, cnt_n, cnt_e, ng_c, xg,
      graph_update_w1, _r2(graph_update_b1),
      graph_update_w2, _r2(graph_update_b2))

    return nout[:N], eout[:E], gout[:N]


# speedup vs baseline: 1.1706x; 1.1706x over previous
import functools

import jax
import jax.numpy as jnp
from jax import lax
from jax.experimental import pallas as pl
from jax.experimental.pallas import tpu as pltpu

NEG_SLOPE = 0.01
TM = 128
TME = 256
F32 = jnp.float32
VMEM_LIMIT = 56 * 1024 * 1024


def _lrelu(x):
    return jnp.where(x > 0, x, NEG_SLOPE * x)


def _round_up(n, m):
    return ((n + m - 1) // m) * m


def _pad_rows(x, rows):
    return jnp.pad(x.astype(F32), ((0, rows - x.shape[0]), (0, 0)))


def _pad_idx(idx, rows, fill):
    return jnp.pad(idx.astype(jnp.int32), (0, rows - idx.shape[0]),
                   constant_values=fill)


def _rep(shape):
    return pl.BlockSpec(shape, lambda *_: (0,) * len(shape))


def _r2(b):
    return b.reshape(1, -1)


def _ff_proj_kernel(xn_ref, xg_ref,
                    wn1, bn1, wn2, bn2,
                    wg1, bg1, wg2, bg2,
                    wps, wpdv, wpdu,
                    vu_ref, pp_ref):
    xn = xn_ref[...]
    hv = _lrelu(jnp.dot(xn, wn1[...], preferred_element_type=F32) + bn1[...])
    v = _lrelu(jnp.dot(hv, wn2[...], preferred_element_type=F32) + bn2[...])
    xg = xg_ref[...]
    hu = _lrelu(jnp.dot(xg, wg1[...], preferred_element_type=F32) + bg1[...])
    u = _lrelu(jnp.dot(hu, wg2[...], preferred_element_type=F32) + bg2[...])
    vu_ref[...] = jnp.concatenate([v, u], axis=1)
    ps = jnp.dot(v, wps[...], preferred_element_type=F32)
    pd = (jnp.dot(v, wpdv[...], preferred_element_type=F32)
          + jnp.dot(u, wpdu[...], preferred_element_type=F32))
    pp_ref[...] = jnp.concatenate([ps, pd], axis=1)


def _edge_kernel(tiles_per_core, n_edges,
                 xe_ref, src_ref, dst_ref, dstp_ref, pp_ref,
                 we1, be1, we2, be2, wee, b1, w2, b2,
                 eout_ref, eagg_ref, gs_scr, gd_scr, pay_scr):
    i = pl.program_id(1)

    @pl.when(i == 0)
    def _():
        eagg_ref[...] = jnp.zeros_like(eagg_ref)
        pay_scr[...] = jnp.zeros(pay_scr.shape, pay_scr.dtype)

    for mi in range(TME):
        d = dstp_ref[0, 0, mi]
        eagg_ref[d, 0] = eagg_ref[d, 0] + pay_scr[mi]

    xe = xe_ref[...]
    h = _lrelu(jnp.dot(xe, we1[...], preferred_element_type=F32) + be1[...])
    e = _lrelu(jnp.dot(h, we2[...], preferred_element_type=F32) + be2[...])
    q = jnp.dot(e, wee[...], preferred_element_type=F32) + b1[...]

    for mi in range(TME):
        gs_scr[mi] = pp_ref[src_ref[0, 0, mi], 0]
        gd_scr[mi] = pp_ref[dst_ref[0, 0, mi], 0]

    hh = gs_scr.shape[1] // 2
    h1 = _lrelu(gs_scr[:, :hh] + gd_scr[:, hh:] + q)
    e_new = _lrelu(jnp.dot(h1, w2[...], preferred_element_type=F32) + b2[...])
    eout_ref[...] = e_new + xe

    ii = jnp.minimum(i, tiles_per_core - 1)
    base = (pl.program_id(0) * tiles_per_core + ii) * TME
    rows = lax.broadcasted_iota(jnp.int32, (TME, 1), 0) + base
    valid = (rows < n_edges).astype(F32)
    pay_scr[...] = jnp.concatenate([e_new, jnp.ones_like(e_new)], axis=1) * valid


def _node_kernel(vu_ref, ega_ref, egb_ref, xn_ref, ng_ref,
                 w_vu, w_ef, b1, w2, b2,
                 nout_ref, pool_ref):
    j = pl.program_id(1)

    @pl.when(j == 0)
    def _():
        pool_ref[...] = jnp.zeros_like(pool_ref)

    o = nout_ref.shape[1]
    agg = ega_ref[...] + egb_ref[...]
    ef_sum = agg[:, :o]
    deg = agg[:, o:]
    ef = ef_sum * pl.reciprocal(jnp.maximum(deg, 1.0), approx=True)
    vu = vu_ref[...]
    h = _lrelu(jnp.dot(vu, w_vu[...], preferred_element_type=F32)
               + jnp.dot(ef, w_ef[...], preferred_element_type=F32)
               + b1[...])
    n_new = _lrelu(jnp.dot(h, w2[...], preferred_element_type=F32) + b2[...])
    nout_ref[...] = n_new + xn_ref[...]

    gp = pool_ref.shape[0]
    u = vu[:, vu.shape[1] // 2:]
    pooled = jnp.concatenate([n_new, ef_sum, u], axis=1)
    row_ids = lax.broadcasted_iota(jnp.int32, (gp, TM), 0)
    oh = (row_ids == ng_ref[...]).astype(F32)
    pool_ref[...] += jnp.dot(oh, pooled, preferred_element_type=F32)


def _graph_kernel(pa_ref, pb_ref, cntn_ref, cnte_ref, ng_ref, xg_ref,
                  w1, b1, w2, b2,
                  gout_ref, gnew_scr):
    j = pl.program_id(1)
    o = gout_ref.shape[1]

    @pl.when(j == 0)
    def _():
        pool = pa_ref[...] + pb_ref[...]
        gp, width = pool.shape
        inv_n = pl.reciprocal(jnp.maximum(cntn_ref[...], 1.0), approx=True)
        inv_e = pl.reciprocal(jnp.maximum(cnte_ref[...], 1.0), approx=True)
        lane = lax.broadcasted_iota(jnp.int32, (gp, width), 1)
        scale = jnp.where(lane < o, inv_n, jnp.where(lane < 2 * o, inv_e, inv_n))
        cat_g = pool * scale
        hg = _lrelu(jnp.dot(cat_g, w1[...], preferred_element_type=F32) + b1[...])
        gnew_scr[...] = _lrelu(jnp.dot(hg, w2[...], preferred_element_type=F32)
                               + b2[...])

    gp = gnew_scr.shape[0]
    col_ids = lax.broadcasted_iota(jnp.int32, (TM, gp), 1)
    oh = (col_ids == ng_ref[...]).astype(F32)
    gout_ref[...] = (jnp.dot(oh, gnew_scr[...], preferred_element_type=F32)
                     + xg_ref[...])


def kernel(ff_node_w1, ff_node_b1, ff_node_w2, ff_node_b2,
           ff_edge_w1, ff_edge_b1, ff_edge_w2, ff_edge_b2,
           ff_graph_w1, ff_graph_b1, ff_graph_w2, ff_graph_b2,
           edge_update_w1, edge_update_b1, edge_update_w2, edge_update_b2,
           node_update_w1, node_update_b1, node_update_w2, node_update_b2,
           graph_update_w1, graph_update_b1, graph_update_w2, graph_update_b2,
           node_feats, edge_feats, graph_feats,
           src, dst, node_graph, batch_num_nodes, batch_num_edges):
    N, D = node_feats.shape
    E = edge_feats.shape[0]
    G = batch_num_nodes.shape[0]
    H = ff_node_w2.shape[1]
    O = edge_update_w2.shape[1]

    Np = _round_up(max(N, 1), 2 * TM)
    Ep = _round_up(max(E, 1), 2 * TME)
    Gp = _round_up(max(G, 1), 8)
    gN = Np // TM
    gN2 = gN // 2
    gE2 = Ep // TME // 2

    w_ps = edge_update_w1[0:H]
    w_pdv = edge_update_w1[H:2 * H]
    w_ee = edge_update_w1[2 * H:3 * H]
    w_pdu = edge_update_w1[3 * H:4 * H]
    w_vu = jnp.concatenate([node_update_w1[0:H],
                            node_update_w1[H + O:H + O + H]], axis=0)
    w_ef = node_update_w1[H:H + O]

    xn = _pad_rows(node_feats, Np)
    xg = _pad_rows(graph_feats, Np)
    xe = _pad_rows(edge_feats, Ep)
    src_t = _pad_idx(src, Ep, 0).reshape(Ep // TME, 1, TME)
    dst_t = _pad_idx(dst, Ep, 0).reshape(Ep // TME, 1, TME)
    ng_r = _pad_idx(node_graph, Np, -1).reshape(1, Np)
    ng_c = _pad_idx(node_graph, Np, -1).reshape(Np, 1)
    cnt_n = jnp.zeros((Gp, 1), F32).at[:G, 0].set(batch_num_nodes.astype(F32))
    cnt_e = jnp.zeros((Gp, 1), F32).at[:G, 0].set(batch_num_edges.astype(F32))

    cp_par = pltpu.CompilerParams(dimension_semantics=("parallel",),
                                  vmem_limit_bytes=VMEM_LIMIT)
    cp_mix = pltpu.CompilerParams(dimension_semantics=("parallel", "arbitrary"),
                                  vmem_limit_bytes=VMEM_LIMIT)

    vu, pp = pl.pallas_call(
        _ff_proj_kernel,
        out_shape=(jax.ShapeDtypeStruct((Np, 2 * H), F32),
                   jax.ShapeDtypeStruct((Np, 2 * H), F32)),
        grid=(gN,),
        in_specs=[
            pl.BlockSpec((TM, D), lambda i: (i, 0)),
            pl.BlockSpec((TM, D), lambda i: (i, 0)),
            _rep((D, D)), _rep((1, D)), _rep((D, H)), _rep((1, H)),
            _rep((D, D)), _rep((1, D)), _rep((D, H)), _rep((1, H)),
            _rep((H, H)), _rep((H, H)), _rep((H, H)),
        ],
        out_specs=(pl.BlockSpec((TM, 2 * H), lambda i: (i, 0)),
                   pl.BlockSpec((TM, 2 * H), lambda i: (i, 0))),
        compiler_params=cp_par,
    )(xn, xg,
      ff_node_w1, _r2(ff_node_b1), ff_node_w2, _r2(ff_node_b2),
      ff_graph_w1, _r2(ff_graph_b1), ff_graph_w2, _r2(ff_graph_b2),
      w_ps, w_pdv, w_pdu)

    pp3 = pp.reshape(Np, 1, 2 * H)
    eout, eagg = pl.pallas_call(
        functools.partial(_edge_kernel, gE2, E),
        out_shape=(jax.ShapeDtypeStruct((Ep, O), F32),
                   jax.ShapeDtypeStruct((2 * Np, 1, 2 * O), F32)),
        grid=(2, gE2 + 1),
        in_specs=[
            pl.BlockSpec((TME, D),
                         lambda c, i, g=gE2: (c * g + jnp.minimum(i, g - 1), 0)),
            pl.BlockSpec((1, 1, TME),
                         lambda c, i, g=gE2: (c * g + jnp.minimum(i, g - 1), 0, 0),
                         memory_space=pltpu.SMEM),
            pl.BlockSpec((1, 1, TME),
                         lambda c, i, g=gE2: (c * g + jnp.minimum(i, g - 1), 0, 0),
                         memory_space=pltpu.SMEM),
            pl.BlockSpec((1, 1, TME),
                         lambda c, i, g=gE2: (c * g + jnp.maximum(i - 1, 0), 0, 0),
                         memory_space=pltpu.SMEM),
            pl.BlockSpec((Np, 1, 2 * H), lambda c, i: (0, 0, 0)),
            _rep((D, D)), _rep((1, D)), _rep((D, H)), _rep((1, H)),
            _rep((H, H)), _rep((1, H)), _rep((H, O)), _rep((1, O)),
        ],
        out_specs=(pl.BlockSpec((TME, O),
                                lambda c, i, g=gE2: (c * g + jnp.minimum(i, g - 1), 0)),
                   pl.BlockSpec((Np, 1, 2 * O), lambda c, i: (c, 0, 0))),
        scratch_shapes=[pltpu.VMEM((TME, 2 * H), F32),
                        pltpu.VMEM((TME, 2 * H), F32),
                        pltpu.VMEM((TME, 2 * O), F32)],
        compiler_params=cp_mix,
    )(xe, src_t, dst_t, dst_t, pp3,
      ff_edge_w1, _r2(ff_edge_b1), ff_edge_w2, _r2(ff_edge_b2),
      w_ee, _r2(edge_update_b1), edge_update_w2, _r2(edge_update_b2))

    eagg2 = eagg.reshape(2 * Np, 2 * O)
    nout, pool = pl.pallas_call(
        _node_kernel,
        out_shape=(jax.ShapeDtypeStruct((Np, O), F32),
                   jax.ShapeDtypeStruct((2 * Gp, 2 * O + H), F32)),
        grid=(2, gN2),
        in_specs=[
            pl.BlockSpec((TM, 2 * H), lambda c, j, g=gN2: (c * g + j, 0)),
            pl.BlockSpec((TM, 2 * O), lambda c, j, g=gN2: (c * g + j, 0)),
            pl.BlockSpec((TM, 2 * O), lambda c, j, g=gN2, n=gN: (c * g + j + n, 0)),
            pl.BlockSpec((TM, D), lambda c, j, g=gN2: (c * g + j, 0)),
            pl.BlockSpec((1, TM), lambda c, j, g=gN2: (0, c * g + j)),
            _rep((2 * H, H)), _rep((O, H)), _rep((1, H)),
            _rep((H, O)), _rep((1, O)),
        ],
        out_specs=(pl.BlockSpec((TM, O), lambda c, j, g=gN2: (c * g + j, 0)),
                   pl.BlockSpec((Gp, 2 * O + H), lambda c, j: (c, 0))),
        compiler_params=cp_mix,
    )(vu, eagg2, eagg2, xn, ng_r,
      w_vu, w_ef, _r2(node_update_b1), node_update_w2, _r2(node_update_b2))

    gout = pl.pallas_call(
        _graph_kernel,
        out_shape=jax.ShapeDtypeStruct((Np, O), F32),
        grid=(2, gN2),
        in_specs=[
            pl.BlockSpec((Gp, 2 * O + H), lambda c, j: (0, 0)),
            pl.BlockSpec((Gp, 2 * O + H), lambda c, j: (1, 0)),
            _rep((Gp, 1)), _rep((Gp, 1)),
            pl.BlockSpec((TM, 1), lambda c, j, g=gN2: (c * g + j, 0)),
            pl.BlockSpec((TM, D), lambda c, j, g=gN2: (c * g + j, 0)),
            _rep((2 * O + H, H)), _rep((1, H)), _rep((H, O)), _rep((1, O)),
        ],
        out_specs=pl.BlockSpec((TM, O), lambda c, j, g=gN2: (c * g + j, 0)),
        scratch_shapes=[pltpu.VMEM((Gp, O), F32)],
        compiler_params=cp_mix,
    )(pool, pool, cnt_n, cnt_e, ng_c, xg,
      graph_update_w1, _r2(graph_update_b1),
      graph_update_w2, _r2(graph_update_b2))

    return nout[:N], eout[:E], gout[:N]


# speedup vs baseline: 1.1723x; 1.0015x over previous
import functools

import jax
import jax.numpy as jnp
from jax import lax
from jax.experimental import pallas as pl
from jax.experimental.pallas import tpu as pltpu

NEG_SLOPE = 0.01
TM = 128
TME = 256
F32 = jnp.float32
VMEM_LIMIT = 56 * 1024 * 1024


def _lrelu(x):
    return jnp.where(x > 0, x, NEG_SLOPE * x)


def _round_up(n, m):
    return ((n + m - 1) // m) * m


def _pad_rows(x, rows):
    return jnp.pad(x.astype(F32), ((0, rows - x.shape[0]), (0, 0)))


def _pad_idx(idx, rows, fill):
    return jnp.pad(idx.astype(jnp.int32), (0, rows - idx.shape[0]),
                   constant_values=fill)


def _rep(shape):
    return pl.BlockSpec(shape, lambda *_: (0,) * len(shape))


def _r2(b):
    return b.reshape(1, -1)


def _ff_proj_kernel(xn_ref, xg_ref,
                    wn1, bn1, wn2, bn2,
                    wg1, bg1, wg2, bg2,
                    wps, wpdv, wpdu,
                    vu_ref, pp_ref):
    xn = xn_ref[...]
    hv = _lrelu(jnp.dot(xn, wn1[...], preferred_element_type=F32) + bn1[...])
    v = _lrelu(jnp.dot(hv, wn2[...], preferred_element_type=F32) + bn2[...])
    xg = xg_ref[...]
    hu = _lrelu(jnp.dot(xg, wg1[...], preferred_element_type=F32) + bg1[...])
    u = _lrelu(jnp.dot(hu, wg2[...], preferred_element_type=F32) + bg2[...])
    vu_ref[...] = jnp.concatenate([v, u], axis=1)
    ps = jnp.dot(v, wps[...], preferred_element_type=F32)
    pd = (jnp.dot(v, wpdv[...], preferred_element_type=F32)
          + jnp.dot(u, wpdu[...], preferred_element_type=F32))
    pp_ref[...] = jnp.concatenate([ps, pd], axis=1)


def _edge_kernel(tiles_per_core, n_edges,
                 xe_ref, src_ref, dst_ref, dstp_ref, pp_ref,
                 we1, be1, we2, be2, wee, b1, w2, b2,
                 eout_ref, eagg_ref, gs_scr, gd_scr, pay_scr):
    i = pl.program_id(1)

    @pl.when(i == 0)
    def _():
        eagg_ref[...] = jnp.zeros_like(eagg_ref)
        pay_scr[...] = jnp.zeros(pay_scr.shape, pay_scr.dtype)

    for mi in range(TME):
        d = dstp_ref[0, 0, mi]
        eagg_ref[d, 0] = eagg_ref[d, 0] + pay_scr[mi]

    xe = xe_ref[...]
    h = _lrelu(jnp.dot(xe, we1[...], preferred_element_type=F32) + be1[...])
    e = _lrelu(jnp.dot(h, we2[...], preferred_element_type=F32) + be2[...])
    q = jnp.dot(e, wee[...], preferred_element_type=F32) + b1[...]

    CH = 8
    for b in range(TME // CH):
        svals = [pp_ref[src_ref[0, 0, b * CH + j], 0] for j in range(CH)]
        dvals = [pp_ref[dst_ref[0, 0, b * CH + j], 0] for j in range(CH)]
        for j in range(CH):
            gs_scr[b * CH + j] = svals[j]
            gd_scr[b * CH + j] = dvals[j]

    hh = gs_scr.shape[1] // 2
    h1 = _lrelu(gs_scr[:, :hh] + gd_scr[:, hh:] + q)
    e_new = _lrelu(jnp.dot(h1, w2[...], preferred_element_type=F32) + b2[...])
    eout_ref[...] = e_new + xe

    ii = jnp.minimum(i, tiles_per_core - 1)
    base = (pl.program_id(0) * tiles_per_core + ii) * TME
    rows = lax.broadcasted_iota(jnp.int32, (TME, 1), 0) + base
    valid = (rows < n_edges).astype(F32)
    pay_scr[...] = jnp.concatenate([e_new, jnp.ones_like(e_new)], axis=1) * valid


def _node_kernel(vu_ref, ega_ref, egb_ref, xn_ref, ng_ref,
                 w_vu, w_ef, b1, w2, b2,
                 nout_ref, pool_ref):
    j = pl.program_id(1)

    @pl.when(j == 0)
    def _():
        pool_ref[...] = jnp.zeros_like(pool_ref)

    o = nout_ref.shape[1]
    agg = ega_ref[...] + egb_ref[...]
    ef_sum = agg[:, :o]
    deg = agg[:, o:]
    ef = ef_sum * pl.reciprocal(jnp.maximum(deg, 1.0), approx=True)
    vu = vu_ref[...]
    h = _lrelu(jnp.dot(vu, w_vu[...], preferred_element_type=F32)
               + jnp.dot(ef, w_ef[...], preferred_element_type=F32)
               + b1[...])
    n_new = _lrelu(jnp.dot(h, w2[...], preferred_element_type=F32) + b2[...])
    nout_ref[...] = n_new + xn_ref[...]

    gp = pool_ref.shape[0]
    u = vu[:, vu.shape[1] // 2:]
    pooled = jnp.concatenate([n_new, ef_sum, u], axis=1)
    row_ids = lax.broadcasted_iota(jnp.int32, (gp, TM), 0)
    oh = (row_ids == ng_ref[...]).astype(F32)
    pool_ref[...] += jnp.dot(oh, pooled, preferred_element_type=F32)


def _graph_kernel(pa_ref, pb_ref, cntn_ref, cnte_ref, ng_ref, xg_ref,
                  w1, b1, w2, b2,
                  gout_ref, gnew_scr):
    j = pl.program_id(1)
    o = gout_ref.shape[1]

    @pl.when(j == 0)
    def _():
        pool = pa_ref[...] + pb_ref[...]
        gp, width = pool.shape
        inv_n = pl.reciprocal(jnp.maximum(cntn_ref[...], 1.0), approx=True)
        inv_e = pl.reciprocal(jnp.maximum(cnte_ref[...], 1.0), approx=True)
        lane = lax.broadcasted_iota(jnp.int32, (gp, width), 1)
        scale = jnp.where(lane < o, inv_n, jnp.where(lane < 2 * o, inv_e, inv_n))
        cat_g = pool * scale
        hg = _lrelu(jnp.dot(cat_g, w1[...], preferred_element_type=F32) + b1[...])
        gnew_scr[...] = _lrelu(jnp.dot(hg, w2[...], preferred_element_type=F32)
                               + b2[...])

    gp = gnew_scr.shape[0]
    col_ids = lax.broadcasted_iota(jnp.int32, (TM, gp), 1)
    oh = (col_ids == ng_ref[...]).astype(F32)
    gout_ref[...] = (jnp.dot(oh, gnew_scr[...], preferred_element_type=F32)
                     + xg_ref[...])


def kernel(ff_node_w1, ff_node_b1, ff_node_w2, ff_node_b2,
           ff_edge_w1, ff_edge_b1, ff_edge_w2, ff_edge_b2,
           ff_graph_w1, ff_graph_b1, ff_graph_w2, ff_graph_b2,
           edge_update_w1, edge_update_b1, edge_update_w2, edge_update_b2,
           node_update_w1, node_update_b1, node_update_w2, node_update_b2,
           graph_update_w1, graph_update_b1, graph_update_w2, graph_update_b2,
           node_feats, edge_feats, graph_feats,
           src, dst, node_graph, batch_num_nodes, batch_num_edges):
    N, D = node_feats.shape
    E = edge_feats.shape[0]
    G = batch_num_nodes.shape[0]
    H = ff_node_w2.shape[1]
    O = edge_update_w2.shape[1]

    Np = _round_up(max(N, 1), 2 * TM)
    Ep = _round_up(max(E, 1), 2 * TME)
    Gp = _round_up(max(G, 1), 8)
    gN = Np // TM
    gN2 = gN // 2
    gE2 = Ep // TME // 2

    w_ps = edge_update_w1[0:H]
    w_pdv = edge_update_w1[H:2 * H]
    w_ee = edge_update_w1[2 * H:3 * H]
    w_pdu = edge_update_w1[3 * H:4 * H]
    w_vu = jnp.concatenate([node_update_w1[0:H],
                            node_update_w1[H + O:H + O + H]], axis=0)
    w_ef = node_update_w1[H:H + O]

    xn = _pad_rows(node_feats, Np)
    xg = _pad_rows(graph_feats, Np)
    xe = _pad_rows(edge_feats, Ep)
    src_t = _pad_idx(src, Ep, 0).reshape(Ep // TME, 1, TME)
    dst_t = _pad_idx(dst, Ep, 0).reshape(Ep // TME, 1, TME)
    ng_r = _pad_idx(node_graph, Np, -1).reshape(1, Np)
    ng_c = _pad_idx(node_graph, Np, -1).reshape(Np, 1)
    cnt_n = jnp.zeros((Gp, 1), F32).at[:G, 0].set(batch_num_nodes.astype(F32))
    cnt_e = jnp.zeros((Gp, 1), F32).at[:G, 0].set(batch_num_edges.astype(F32))

    cp_par = pltpu.CompilerParams(dimension_semantics=("parallel",),
                                  vmem_limit_bytes=VMEM_LIMIT)
    cp_mix = pltpu.CompilerParams(dimension_semantics=("parallel", "arbitrary"),
                                  vmem_limit_bytes=VMEM_LIMIT)

    vu, pp = pl.pallas_call(
        _ff_proj_kernel,
        out_shape=(jax.ShapeDtypeStruct((Np, 2 * H), F32),
                   jax.ShapeDtypeStruct((Np, 2 * H), F32)),
        grid=(gN,),
        in_specs=[
            pl.BlockSpec((TM, D), lambda i: (i, 0)),
            pl.BlockSpec((TM, D), lambda i: (i, 0)),
            _rep((D, D)), _rep((1, D)), _rep((D, H)), _rep((1, H)),
            _rep((D, D)), _rep((1, D)), _rep((D, H)), _rep((1, H)),
            _rep((H, H)), _rep((H, H)), _rep((H, H)),
        ],
        out_specs=(pl.BlockSpec((TM, 2 * H), lambda i: (i, 0)),
                   pl.BlockSpec((TM, 2 * H), lambda i: (i, 0))),
        compiler_params=cp_par,
    )(xn, xg,
      ff_node_w1, _r2(ff_node_b1), ff_node_w2, _r2(ff_node_b2),
      ff_graph_w1, _r2(ff_graph_b1), ff_graph_w2, _r2(ff_graph_b2),
      w_ps, w_pdv, w_pdu)

    pp3 = pp.reshape(Np, 1, 2 * H)
    eout, eagg = pl.pallas_call(
        functools.partial(_edge_kernel, gE2, E),
        out_shape=(jax.ShapeDtypeStruct((Ep, O), F32),
                   jax.ShapeDtypeStruct((2 * Np, 1, 2 * O), F32)),
        grid=(2, gE2 + 1),
        in_specs=[
            pl.BlockSpec((TME, D),
                         lambda c, i, g=gE2: (c * g + jnp.minimum(i, g - 1), 0)),
            pl.BlockSpec((1, 1, TME),
                         lambda c, i, g=gE2: (c * g + jnp.minimum(i, g - 1), 0, 0),
                         memory_space=pltpu.SMEM),
            pl.BlockSpec((1, 1, TME),
                         lambda c, i, g=gE2: (c * g + jnp.minimum(i, g - 1), 0, 0),
                         memory_space=pltpu.SMEM),
            pl.BlockSpec((1, 1, TME),
                         lambda c, i, g=gE2: (c * g + jnp.maximum(i - 1, 0), 0, 0),
                         memory_space=pltpu.SMEM),
            pl.BlockSpec((Np, 1, 2 * H), lambda c, i: (0, 0, 0)),
            _rep((D, D)), _rep((1, D)), _rep((D, H)), _rep((1, H)),
            _rep((H, H)), _rep((1, H)), _rep((H, O)), _rep((1, O)),
        ],
        out_specs=(pl.BlockSpec((TME, O),
                                lambda c, i, g=gE2: (c * g + jnp.minimum(i, g - 1), 0)),
                   pl.BlockSpec((Np, 1, 2 * O), lambda c, i: (c, 0, 0))),
        scratch_shapes=[pltpu.VMEM((TME, 2 * H), F32),
                        pltpu.VMEM((TME, 2 * H), F32),
                        pltpu.VMEM((TME, 2 * O), F32)],
        compiler_params=cp_mix,
    )(xe, src_t, dst_t, dst_t, pp3,
      ff_edge_w1, _r2(ff_edge_b1), ff_edge_w2, _r2(ff_edge_b2),
      w_ee, _r2(edge_update_b1), edge_update_w2, _r2(edge_update_b2))

    eagg2 = eagg.reshape(2 * Np, 2 * O)
    nout, pool = pl.pallas_call(
        _node_kernel,
        out_shape=(jax.ShapeDtypeStruct((Np, O), F32),
                   jax.ShapeDtypeStruct((2 * Gp, 2 * O + H), F32)),
        grid=(2, gN2),
        in_specs=[
            pl.BlockSpec((TM, 2 * H), lambda c, j, g=gN2: (c * g + j, 0)),
            pl.BlockSpec((TM, 2 * O), lambda c, j, g=gN2: (c * g + j, 0)),
            pl.BlockSpec((TM, 2 * O), lambda c, j, g=gN2, n=gN: (c * g + j + n, 0)),
            pl.BlockSpec((TM, D), lambda c, j, g=gN2: (c * g + j, 0)),
            pl.BlockSpec((1, TM), lambda c, j, g=gN2: (0, c * g + j)),
            _rep((2 * H, H)), _rep((O, H)), _rep((1, H)),
            _rep((H, O)), _rep((1, O)),
        ],
        out_specs=(pl.BlockSpec((TM, O), lambda c, j, g=gN2: (c * g + j, 0)),
                   pl.BlockSpec((Gp, 2 * O + H), lambda c, j: (c, 0))),
        compiler_params=cp_mix,
    )(vu, eagg2, eagg2, xn, ng_r,
      w_vu, w_ef, _r2(node_update_b1), node_update_w2, _r2(node_update_b2))

    gout = pl.pallas_call(
        _graph_kernel,
        out_shape=jax.ShapeDtypeStruct((Np, O), F32),
        grid=(2, gN2),
        in_specs=[
            pl.BlockSpec((Gp, 2 * O + H), lambda c, j: (0, 0)),
            pl.BlockSpec((Gp, 2 * O + H), lambda c, j: (1, 0)),
            _rep((Gp, 1)), _rep((Gp, 1)),
            pl.BlockSpec((TM, 1), lambda c, j, g=gN2: (c * g + j, 0)),
            pl.BlockSpec((TM, D), lambda c, j, g=gN2: (c * g + j, 0)),
            _rep((2 * O + H, H)), _rep((1, H)), _rep((H, O)), _rep((1, O)),
        ],
        out_specs=pl.BlockSpec((TM, O), lambda c, j, g=gN2: (c * g + j, 0)),
        scratch_shapes=[pltpu.VMEM((Gp, O), F32)],
        compiler_params=cp_mix,
    )(pool, pool, cnt_n, cnt_e, ng_c, xg,
      graph_update_w1, _r2(graph_update_b1),
      graph_update_w2, _r2(graph_update_b2))

    return nout[:N], eout[:E], gout[:N]


# speedup vs baseline: 1.1804x; 1.0069x over previous
import functools

import jax
import jax.numpy as jnp
from jax import lax
from jax.experimental import pallas as pl
from jax.experimental.pallas import tpu as pltpu

NEG_SLOPE = 0.01
TM = 128
TME = 512
F32 = jnp.float32
VMEM_LIMIT = 56 * 1024 * 1024


def _lrelu(x):
    return jnp.where(x > 0, x, NEG_SLOPE * x)


def _round_up(n, m):
    return ((n + m - 1) // m) * m


def _pad_rows(x, rows):
    return jnp.pad(x.astype(F32), ((0, rows - x.shape[0]), (0, 0)))


def _pad_idx(idx, rows, fill):
    return jnp.pad(idx.astype(jnp.int32), (0, rows - idx.shape[0]),
                   constant_values=fill)


def _rep(shape):
    return pl.BlockSpec(shape, lambda *_: (0,) * len(shape))


def _r2(b):
    return b.reshape(1, -1)


def _ff_proj_kernel(xn_ref, xg_ref,
                    wn1, bn1, wn2, bn2,
                    wg1, bg1, wg2, bg2,
                    wps, wpdv, wpdu,
                    vu_ref, pp_ref):
    xn = xn_ref[...]
    hv = _lrelu(jnp.dot(xn, wn1[...], preferred_element_type=F32) + bn1[...])
    v = _lrelu(jnp.dot(hv, wn2[...], preferred_element_type=F32) + bn2[...])
    xg = xg_ref[...]
    hu = _lrelu(jnp.dot(xg, wg1[...], preferred_element_type=F32) + bg1[...])
    u = _lrelu(jnp.dot(hu, wg2[...], preferred_element_type=F32) + bg2[...])
    vu_ref[...] = jnp.concatenate([v, u], axis=1)
    ps = jnp.dot(v, wps[...], preferred_element_type=F32)
    pd = (jnp.dot(v, wpdv[...], preferred_element_type=F32)
          + jnp.dot(u, wpdu[...], preferred_element_type=F32))
    pp_ref[...] = jnp.concatenate([ps, pd], axis=1)


def _edge_kernel(tiles_per_core, n_edges,
                 xe_ref, src_ref, dst_ref, dstp_ref, pp_ref,
                 we1, be1, we2, be2, wee, b1, w2, b2,
                 eout_ref, eagg_ref, gs_scr, gd_scr, pay_scr):
    i = pl.program_id(1)

    @pl.when(i == 0)
    def _():
        nrows = eagg_ref.shape[0]
        zblk = jnp.zeros((128,) + eagg_ref.shape[1:], eagg_ref.dtype)

        def _zero(k, carry):
            eagg_ref[pl.ds(k * 128, 128)] = zblk
            return carry

        lax.fori_loop(0, nrows // 128, _zero, 0)
        pay_scr[...] = jnp.zeros(pay_scr.shape, pay_scr.dtype)

    for mi in range(TME):
        d = dstp_ref[0, 0, mi]
        eagg_ref[d, 0] = eagg_ref[d, 0] + pay_scr[mi]

    xe = xe_ref[...]
    h = _lrelu(jnp.dot(xe, we1[...], preferred_element_type=F32) + be1[...])
    e = _lrelu(jnp.dot(h, we2[...], preferred_element_type=F32) + be2[...])
    q = jnp.dot(e, wee[...], preferred_element_type=F32) + b1[...]

    CH = 8
    for b in range(TME // CH):
        svals = [pp_ref[src_ref[0, 0, b * CH + j], 0] for j in range(CH)]
        dvals = [pp_ref[dst_ref[0, 0, b * CH + j], 0] for j in range(CH)]
        for j in range(CH):
            gs_scr[b * CH + j] = svals[j]
            gd_scr[b * CH + j] = dvals[j]

    hh = gs_scr.shape[1] // 2
    h1 = _lrelu(gs_scr[:, :hh] + gd_scr[:, hh:] + q)
    e_new = _lrelu(jnp.dot(h1, w2[...], preferred_element_type=F32) + b2[...])
    eout_ref[...] = e_new + xe

    ii = jnp.minimum(i, tiles_per_core - 1)
    base = (pl.program_id(0) * tiles_per_core + ii) * TME
    rows = lax.broadcasted_iota(jnp.int32, (TME, 1), 0) + base
    valid = (rows < n_edges).astype(F32)
    pay_scr[...] = jnp.concatenate([e_new, jnp.ones_like(e_new)], axis=1) * valid


def _node_kernel(vu_ref, ega_ref, egb_ref, xn_ref, ng_ref,
                 w_vu, w_ef, b1, w2, b2,
                 nout_ref, pool_ref):
    j = pl.program_id(1)

    @pl.when(j == 0)
    def _():
        pool_ref[...] = jnp.zeros_like(pool_ref)

    o = nout_ref.shape[1]
    agg = ega_ref[...] + egb_ref[...]
    ef_sum = agg[:, :o]
    deg = agg[:, o:]
    ef = ef_sum * pl.reciprocal(jnp.maximum(deg, 1.0), approx=True)
    vu = vu_ref[...]
    h = _lrelu(jnp.dot(vu, w_vu[...], preferred_element_type=F32)
               + jnp.dot(ef, w_ef[...], preferred_element_type=F32)
               + b1[...])
    n_new = _lrelu(jnp.dot(h, w2[...], preferred_element_type=F32) + b2[...])
    nout_ref[...] = n_new + xn_ref[...]

    gp = pool_ref.shape[0]
    u = vu[:, vu.shape[1] // 2:]
    pooled = jnp.concatenate([n_new, ef_sum, u], axis=1)
    row_ids = lax.broadcasted_iota(jnp.int32, (gp, TM), 0)
    oh = (row_ids == ng_ref[...]).astype(F32)
    pool_ref[...] += jnp.dot(oh, pooled, preferred_element_type=F32)


def _graph_kernel(pa_ref, pb_ref, cntn_ref, cnte_ref, ng_ref, xg_ref,
                  w1, b1, w2, b2,
                  gout_ref, gnew_scr):
    j = pl.program_id(1)
    o = gout_ref.shape[1]

    @pl.when(j == 0)
    def _():
        pool = pa_ref[...] + pb_ref[...]
        gp, width = pool.shape
        inv_n = pl.reciprocal(jnp.maximum(cntn_ref[...], 1.0), approx=True)
        inv_e = pl.reciprocal(jnp.maximum(cnte_ref[...], 1.0), approx=True)
        lane = lax.broadcasted_iota(jnp.int32, (gp, width), 1)
        scale = jnp.where(lane < o, inv_n, jnp.where(lane < 2 * o, inv_e, inv_n))
        cat_g = pool * scale
        hg = _lrelu(jnp.dot(cat_g, w1[...], preferred_element_type=F32) + b1[...])
        gnew_scr[...] = _lrelu(jnp.dot(hg, w2[...], preferred_element_type=F32)
                               + b2[...])

    gp = gnew_scr.shape[0]
    col_ids = lax.broadcasted_iota(jnp.int32, (TM, gp), 1)
    oh = (col_ids == ng_ref[...]).astype(F32)
    gout_ref[...] = (jnp.dot(oh, gnew_scr[...], preferred_element_type=F32)
                     + xg_ref[...])


def kernel(ff_node_w1, ff_node_b1, ff_node_w2, ff_node_b2,
           ff_edge_w1, ff_edge_b1, ff_edge_w2, ff_edge_b2,
           ff_graph_w1, ff_graph_b1, ff_graph_w2, ff_graph_b2,
           edge_update_w1, edge_update_b1, edge_update_w2, edge_update_b2,
           node_update_w1, node_update_b1, node_update_w2, node_update_b2,
           graph_update_w1, graph_update_b1, graph_update_w2, graph_update_b2,
           node_feats, edge_feats, graph_feats,
           src, dst, node_graph, batch_num_nodes, batch_num_edges):
    N, D = node_feats.shape
    E = edge_feats.shape[0]
    G = batch_num_nodes.shape[0]
    H = ff_node_w2.shape[1]
    O = edge_update_w2.shape[1]

    Np = _round_up(max(N, 1), 2 * TM)
    Ep = _round_up(max(E, 1), 2 * TME)
    Gp = _round_up(max(G, 1), 8)
    gN = Np // TM
    gN2 = gN // 2
    gE2 = Ep // TME // 2

    w_ps = edge_update_w1[0:H]
    w_pdv = edge_update_w1[H:2 * H]
    w_ee = edge_update_w1[2 * H:3 * H]
    w_pdu = edge_update_w1[3 * H:4 * H]
    w_vu = jnp.concatenate([node_update_w1[0:H],
                            node_update_w1[H + O:H + O + H]], axis=0)
    w_ef = node_update_w1[H:H + O]

    xn = _pad_rows(node_feats, Np)
    xg = _pad_rows(graph_feats, Np)
    xe = _pad_rows(edge_feats, Ep)
    src_t = _pad_idx(src, Ep, 0).reshape(Ep // TME, 1, TME)
    dst_t = _pad_idx(dst, Ep, 0).reshape(Ep // TME, 1, TME)
    ng_r = _pad_idx(node_graph, Np, -1).reshape(1, Np)
    ng_c = _pad_idx(node_graph, Np, -1).reshape(Np, 1)
    cnt_n = jnp.zeros((Gp, 1), F32).at[:G, 0].set(batch_num_nodes.astype(F32))
    cnt_e = jnp.zeros((Gp, 1), F32).at[:G, 0].set(batch_num_edges.astype(F32))

    cp_par = pltpu.CompilerParams(dimension_semantics=("parallel",),
                                  vmem_limit_bytes=VMEM_LIMIT)
    cp_mix = pltpu.CompilerParams(dimension_semantics=("parallel", "arbitrary"),
                                  vmem_limit_bytes=VMEM_LIMIT)

    vu, pp = pl.pallas_call(
        _ff_proj_kernel,
        out_shape=(jax.ShapeDtypeStruct((Np, 2 * H), F32),
                   jax.ShapeDtypeStruct((Np, 2 * H), F32)),
        grid=(gN,),
        in_specs=[
            pl.BlockSpec((TM, D), lambda i: (i, 0)),
            pl.BlockSpec((TM, D), lambda i: (i, 0)),
            _rep((D, D)), _rep((1, D)), _rep((D, H)), _rep((1, H)),
            _rep((D, D)), _rep((1, D)), _rep((D, H)), _rep((1, H)),
            _rep((H, H)), _rep((H, H)), _rep((H, H)),
        ],
        out_specs=(pl.BlockSpec((TM, 2 * H), lambda i: (i, 0)),
                   pl.BlockSpec((TM, 2 * H), lambda i: (i, 0))),
        compiler_params=cp_par,
    )(xn, xg,
      ff_node_w1, _r2(ff_node_b1), ff_node_w2, _r2(ff_node_b2),
      ff_graph_w1, _r2(ff_graph_b1), ff_graph_w2, _r2(ff_graph_b2),
      w_ps, w_pdv, w_pdu)

    pp3 = pp.reshape(Np, 1, 2 * H)
    eout, eagg = pl.pallas_call(
        functools.partial(_edge_kernel, gE2, E),
        out_shape=(jax.ShapeDtypeStruct((Ep, O), F32),
                   jax.ShapeDtypeStruct((2 * Np, 1, 2 * O), F32)),
        grid=(2, gE2 + 1),
        in_specs=[
            pl.BlockSpec((TME, D),
                         lambda c, i, g=gE2: (c * g + jnp.minimum(i, g - 1), 0)),
            pl.BlockSpec((1, 1, TME),
                         lambda c, i, g=gE2: (c * g + jnp.minimum(i, g - 1), 0, 0),
                         memory_space=pltpu.SMEM),
            pl.BlockSpec((1, 1, TME),
                         lambda c, i, g=gE2: (c * g + jnp.minimum(i, g - 1), 0, 0),
                         memory_space=pltpu.SMEM),
            pl.BlockSpec((1, 1, TME),
                         lambda c, i, g=gE2: (c * g + jnp.maximum(i - 1, 0), 0, 0),
                         memory_space=pltpu.SMEM),
            pl.BlockSpec((Np, 1, 2 * H), lambda c, i: (0, 0, 0)),
            _rep((D, D)), _rep((1, D)), _rep((D, H)), _rep((1, H)),
            _rep((H, H)), _rep((1, H)), _rep((H, O)), _rep((1, O)),
        ],
        out_specs=(pl.BlockSpec((TME, O),
                                lambda c, i, g=gE2: (c * g + jnp.minimum(i, g - 1), 0)),
                   pl.BlockSpec((Np, 1, 2 * O), lambda c, i: (c, 0, 0))),
        scratch_shapes=[pltpu.VMEM((TME, 2 * H), F32),
                        pltpu.VMEM((TME, 2 * H), F32),
                        pltpu.VMEM((TME, 2 * O), F32)],
        compiler_params=cp_mix,
    )(xe, src_t, dst_t, dst_t, pp3,
      ff_edge_w1, _r2(ff_edge_b1), ff_edge_w2, _r2(ff_edge_b2),
      w_ee, _r2(edge_update_b1), edge_update_w2, _r2(edge_update_b2))

    eagg2 = eagg.reshape(2 * Np, 2 * O)
    nout, pool = pl.pallas_call(
        _node_kernel,
        out_shape=(jax.ShapeDtypeStruct((Np, O), F32),
                   jax.ShapeDtypeStruct((2 * Gp, 2 * O + H), F32)),
        grid=(2, gN2),
        in_specs=[
            pl.BlockSpec((TM, 2 * H), lambda c, j, g=gN2: (c * g + j, 0)),
            pl.BlockSpec((TM, 2 * O), lambda c, j, g=gN2: (c * g + j, 0)),
            pl.BlockSpec((TM, 2 * O), lambda c, j, g=gN2, n=gN: (c * g + j + n, 0)),
            pl.BlockSpec((TM, D), lambda c, j, g=gN2: (c * g + j, 0)),
            pl.BlockSpec((1, TM), lambda c, j, g=gN2: (0, c * g + j)),
            _rep((2 * H, H)), _rep((O, H)), _rep((1, H)),
            _rep((H, O)), _rep((1, O)),
        ],
        out_specs=(pl.BlockSpec((TM, O), lambda c, j, g=gN2: (c * g + j, 0)),
                   pl.BlockSpec((Gp, 2 * O + H), lambda c, j: (c, 0))),
        compiler_params=cp_mix,
    )(vu, eagg2, eagg2, xn, ng_r,
      w_vu, w_ef, _r2(node_update_b1), node_update_w2, _r2(node_update_b2))

    gout = pl.pallas_call(
        _graph_kernel,
        out_shape=jax.ShapeDtypeStruct((Np, O), F32),
        grid=(2, gN2),
        in_specs=[
            pl.BlockSpec((Gp, 2 * O + H), lambda c, j: (0, 0)),
            pl.BlockSpec((Gp, 2 * O + H), lambda c, j: (1, 0)),
            _rep((Gp, 1)), _rep((Gp, 1)),
            pl.BlockSpec((TM, 1), lambda c, j, g=gN2: (c * g + j, 0)),
            pl.BlockSpec((TM, D), lambda c, j, g=gN2: (c * g + j, 0)),
            _rep((2 * O + H, H)), _rep((1, H)), _rep((H, O)), _rep((1, O)),
        ],
        out_specs=pl.BlockSpec((TM, O), lambda c, j, g=gN2: (c * g + j, 0)),
        scratch_shapes=[pltpu.VMEM((Gp, O), F32)],
        compiler_params=cp_mix,
    )(pool, pool, cnt_n, cnt_e, ng_c, xg,
      graph_update_w1, _r2(graph_update_b1),
      graph_update_w2, _r2(graph_update_b2))

    return nout[:N], eout[:E], gout[:N]


# speedup vs baseline: 1.2059x; 1.0216x over previous
import functools

import jax
import jax.numpy as jnp
from jax import lax
from jax.experimental import pallas as pl
from jax.experimental.pallas import tpu as pltpu

NEG_SLOPE = 0.01
TM = 128
TME = 512
F32 = jnp.float32
VMEM_LIMIT = 56 * 1024 * 1024


def _lrelu(x):
    return jnp.where(x > 0, x, NEG_SLOPE * x)


def _round_up(n, m):
    return ((n + m - 1) // m) * m


def _pad_rows(x, rows):
    return jnp.pad(x.astype(F32), ((0, rows - x.shape[0]), (0, 0)))


def _pad_idx(idx, rows, fill):
    return jnp.pad(idx.astype(jnp.int32), (0, rows - idx.shape[0]),
                   constant_values=fill)


def _rep(shape):
    return pl.BlockSpec(shape, lambda *_: (0,) * len(shape))


def _r2(b):
    return b.reshape(1, -1)


def _ff_proj_kernel(xn_ref, xg_ref,
                    wn1, bn1, wn2, bn2,
                    wg1, bg1, wg2, bg2,
                    wps, wpdv, wpdu,
                    vu_ref, pp_ref):
    xn = xn_ref[...]
    hv = _lrelu(jnp.dot(xn, wn1[...], preferred_element_type=F32) + bn1[...])
    v = _lrelu(jnp.dot(hv, wn2[...], preferred_element_type=F32) + bn2[...])
    xg = xg_ref[...]
    hu = _lrelu(jnp.dot(xg, wg1[...], preferred_element_type=F32) + bg1[...])
    u = _lrelu(jnp.dot(hu, wg2[...], preferred_element_type=F32) + bg2[...])
    vu_ref[...] = jnp.concatenate([v, u], axis=1)
    ps = jnp.dot(v, wps[...], preferred_element_type=F32)
    pd = (jnp.dot(v, wpdv[...], preferred_element_type=F32)
          + jnp.dot(u, wpdu[...], preferred_element_type=F32))
    pp_ref[...] = jnp.concatenate([ps, pd], axis=1)


def _edge_kernel(n_tiles, n_edges,
                 xe_ref, src_ref, dst_ref, dstp_ref, pp_ref,
                 we1, be1, we2, be2, wee, b1, w2, b2,
                 eout_ref, eagg_ref, gs_scr, gd_scr, pay_scr):
    i = pl.program_id(0)

    @pl.when(i == 0)
    def _():
        nrows = eagg_ref.shape[0]
        zblk = jnp.zeros((128,) + eagg_ref.shape[1:], eagg_ref.dtype)

        def _zero(k, carry):
            eagg_ref[pl.ds(k * 128, 128)] = zblk
            return carry

        lax.fori_loop(0, nrows // 128, _zero, 0)
        pay_scr[...] = jnp.zeros(pay_scr.shape, pay_scr.dtype)

    for mi in range(TME):
        d = dstp_ref[0, 0, mi]
        eagg_ref[d, 0] = eagg_ref[d, 0] + pay_scr[mi]

    xe = xe_ref[...]
    h = _lrelu(jnp.dot(xe, we1[...], preferred_element_type=F32) + be1[...])
    e = _lrelu(jnp.dot(h, we2[...], preferred_element_type=F32) + be2[...])
    q = jnp.dot(e, wee[...], preferred_element_type=F32) + b1[...]

    CH = 8
    for b in range(TME // CH):
        svals = [pp_ref[src_ref[0, 0, b * CH + j], 0] for j in range(CH)]
        dvals = [pp_ref[dst_ref[0, 0, b * CH + j], 0] for j in range(CH)]
        for j in range(CH):
            gs_scr[b * CH + j] = svals[j]
            gd_scr[b * CH + j] = dvals[j]

    hh = gs_scr.shape[1] // 2
    h1 = _lrelu(gs_scr[:, :hh] + gd_scr[:, hh:] + q)
    e_new = _lrelu(jnp.dot(h1, w2[...], preferred_element_type=F32) + b2[...])
    eout_ref[...] = e_new + xe

    base = jnp.minimum(i, n_tiles - 1) * TME
    rows = lax.broadcasted_iota(jnp.int32, (TME, 1), 0) + base
    valid = (rows < n_edges).astype(F32)
    pay_scr[...] = jnp.concatenate([e_new, jnp.ones_like(e_new)], axis=1) * valid


def _node_kernel(vu_ref, eg_ref, xn_ref, ng_ref,
                 w_vu, w_ef, b1, w2, b2,
                 nout_ref, pool_ref):
    j = pl.program_id(0)

    @pl.when(j == 0)
    def _():
        pool_ref[...] = jnp.zeros_like(pool_ref)

    o = nout_ref.shape[1]
    agg = eg_ref[...]
    ef_sum = agg[:, :o]
    deg = agg[:, o:]
    ef = ef_sum * pl.reciprocal(jnp.maximum(deg, 1.0), approx=True)
    vu = vu_ref[...]
    h = _lrelu(jnp.dot(vu, w_vu[...], preferred_element_type=F32)
               + jnp.dot(ef, w_ef[...], preferred_element_type=F32)
               + b1[...])
    n_new = _lrelu(jnp.dot(h, w2[...], preferred_element_type=F32) + b2[...])
    nout_ref[...] = n_new + xn_ref[...]

    gp = pool_ref.shape[0]
    u = vu[:, vu.shape[1] // 2:]
    pooled = jnp.concatenate([n_new, ef_sum, u], axis=1)
    row_ids = lax.broadcasted_iota(jnp.int32, (gp, TM), 0)
    oh = (row_ids == ng_ref[...]).astype(F32)
    pool_ref[...] += jnp.dot(oh, pooled, preferred_element_type=F32)


def _graph_kernel(pool_ref, cntn_ref, cnte_ref, ng_ref, xg_ref,
                  w1, b1, w2, b2,
                  gout_ref, gnew_scr):
    j = pl.program_id(0)
    o = gout_ref.shape[1]

    @pl.when(j == 0)
    def _():
        pool = pool_ref[...]
        gp, width = pool.shape
        inv_n = pl.reciprocal(jnp.maximum(cntn_ref[...], 1.0), approx=True)
        inv_e = pl.reciprocal(jnp.maximum(cnte_ref[...], 1.0), approx=True)
        lane = lax.broadcasted_iota(jnp.int32, (gp, width), 1)
        scale = jnp.where(lane < o, inv_n, jnp.where(lane < 2 * o, inv_e, inv_n))
        cat_g = pool * scale
        hg = _lrelu(jnp.dot(cat_g, w1[...], preferred_element_type=F32) + b1[...])
        gnew_scr[...] = _lrelu(jnp.dot(hg, w2[...], preferred_element_type=F32)
                               + b2[...])

    gp = gnew_scr.shape[0]
    col_ids = lax.broadcasted_iota(jnp.int32, (TM, gp), 1)
    oh = (col_ids == ng_ref[...]).astype(F32)
    gout_ref[...] = (jnp.dot(oh, gnew_scr[...], preferred_element_type=F32)
                     + xg_ref[...])


def kernel(ff_node_w1, ff_node_b1, ff_node_w2, ff_node_b2,
           ff_edge_w1, ff_edge_b1, ff_edge_w2, ff_edge_b2,
           ff_graph_w1, ff_graph_b1, ff_graph_w2, ff_graph_b2,
           edge_update_w1, edge_update_b1, edge_update_w2, edge_update_b2,
           node_update_w1, node_update_b1, node_update_w2, node_update_b2,
           graph_update_w1, graph_update_b1, graph_update_w2, graph_update_b2,
           node_feats, edge_feats, graph_feats,
           src, dst, node_graph, batch_num_nodes, batch_num_edges):
    N, D = node_feats.shape
    E = edge_feats.shape[0]
    G = batch_num_nodes.shape[0]
    H = ff_node_w2.shape[1]
    O = edge_update_w2.shape[1]

    Np = _round_up(max(N, 1), 2 * TM)
    Ep = _round_up(max(E, 1), 2 * TME)
    Gp = _round_up(max(G, 1), 8)
    gN = Np // TM
    gE = Ep // TME

    w_ps = edge_update_w1[0:H]
    w_pdv = edge_update_w1[H:2 * H]
    w_ee = edge_update_w1[2 * H:3 * H]
    w_pdu = edge_update_w1[3 * H:4 * H]
    w_vu = jnp.concatenate([node_update_w1[0:H],
                            node_update_w1[H + O:H + O + H]], axis=0)
    w_ef = node_update_w1[H:H + O]

    xn = _pad_rows(node_feats, Np)
    xg = _pad_rows(graph_feats, Np)
    xe = _pad_rows(edge_feats, Ep)
    src_t = _pad_idx(src, Ep, 0).reshape(Ep // TME, 1, TME)
    dst_t = _pad_idx(dst, Ep, 0).reshape(Ep // TME, 1, TME)
    ng_r = _pad_idx(node_graph, Np, -1).reshape(1, Np)
    ng_c = _pad_idx(node_graph, Np, -1).reshape(Np, 1)
    cnt_n = jnp.zeros((Gp, 1), F32).at[:G, 0].set(batch_num_nodes.astype(F32))
    cnt_e = jnp.zeros((Gp, 1), F32).at[:G, 0].set(batch_num_edges.astype(F32))

    cp_arb = pltpu.CompilerParams(dimension_semantics=("arbitrary",),
                                  vmem_limit_bytes=VMEM_LIMIT)

    vu, pp = pl.pallas_call(
        _ff_proj_kernel,
        out_shape=(jax.ShapeDtypeStruct((Np, 2 * H), F32),
                   jax.ShapeDtypeStruct((Np, 2 * H), F32)),
        grid=(gN,),
        in_specs=[
            pl.BlockSpec((TM, D), lambda i: (i, 0)),
            pl.BlockSpec((TM, D), lambda i: (i, 0)),
            _rep((D, D)), _rep((1, D)), _rep((D, H)), _rep((1, H)),
            _rep((D, D)), _rep((1, D)), _rep((D, H)), _rep((1, H)),
            _rep((H, H)), _rep((H, H)), _rep((H, H)),
        ],
        out_specs=(pl.BlockSpec((TM, 2 * H), lambda i: (i, 0)),
                   pl.BlockSpec((TM, 2 * H), lambda i: (i, 0))),
        compiler_params=cp_arb,
    )(xn, xg,
      ff_node_w1, _r2(ff_node_b1), ff_node_w2, _r2(ff_node_b2),
      ff_graph_w1, _r2(ff_graph_b1), ff_graph_w2, _r2(ff_graph_b2),
      w_ps, w_pdv, w_pdu)

    pp3 = pp.reshape(Np, 1, 2 * H)
    eout, eagg = pl.pallas_call(
        functools.partial(_edge_kernel, gE, E),
        out_shape=(jax.ShapeDtypeStruct((Ep, O), F32),
                   jax.ShapeDtypeStruct((Np, 1, 2 * O), F32)),
        grid=(gE + 1,),
        in_specs=[
            pl.BlockSpec((TME, D),
                         lambda i, g=gE: (jnp.minimum(i, g - 1), 0)),
            pl.BlockSpec((1, 1, TME),
                         lambda i, g=gE: (jnp.minimum(i, g - 1), 0, 0),
                         memory_space=pltpu.SMEM),
            pl.BlockSpec((1, 1, TME),
                         lambda i, g=gE: (jnp.minimum(i, g - 1), 0, 0),
                         memory_space=pltpu.SMEM),
            pl.BlockSpec((1, 1, TME),
                         lambda i: (jnp.maximum(i - 1, 0), 0, 0),
                         memory_space=pltpu.SMEM),
            pl.BlockSpec((Np, 1, 2 * H), lambda i: (0, 0, 0)),
            _rep((D, D)), _rep((1, D)), _rep((D, H)), _rep((1, H)),
            _rep((H, H)), _rep((1, H)), _rep((H, O)), _rep((1, O)),
        ],
        out_specs=(pl.BlockSpec((TME, O),
                                lambda i, g=gE: (jnp.minimum(i, g - 1), 0)),
                   pl.BlockSpec((Np, 1, 2 * O), lambda i: (0, 0, 0))),
        scratch_shapes=[pltpu.VMEM((TME, 2 * H), F32),
                        pltpu.VMEM((TME, 2 * H), F32),
                        pltpu.VMEM((TME, 2 * O), F32)],
        compiler_params=cp_arb,
    )(xe, src_t, dst_t, dst_t, pp3,
      ff_edge_w1, _r2(ff_edge_b1), ff_edge_w2, _r2(ff_edge_b2),
      w_ee, _r2(edge_update_b1), edge_update_w2, _r2(edge_update_b2))

    eagg2 = eagg.reshape(Np, 2 * O)
    nout, pool = pl.pallas_call(
        _node_kernel,
        out_shape=(jax.ShapeDtypeStruct((Np, O), F32),
                   jax.ShapeDtypeStruct((Gp, 2 * O + H), F32)),
        grid=(gN,),
        in_specs=[
            pl.BlockSpec((TM, 2 * H), lambda j: (j, 0)),
            pl.BlockSpec((TM, 2 * O), lambda j: (j, 0)),
            pl.BlockSpec((TM, D), lambda j: (j, 0)),
            pl.BlockSpec((1, TM), lambda j: (0, j)),
            _rep((2 * H, H)), _rep((O, H)), _rep((1, H)),
            _rep((H, O)), _rep((1, O)),
        ],
        out_specs=(pl.BlockSpec((TM, O), lambda j: (j, 0)),
                   pl.BlockSpec((Gp, 2 * O + H), lambda j: (0, 0))),
        compiler_params=cp_arb,
    )(vu, eagg2, xn, ng_r,
      w_vu, w_ef, _r2(node_update_b1), node_update_w2, _r2(node_update_b2))

    gout = pl.pallas_call(
        _graph_kernel,
        out_shape=jax.ShapeDtypeStruct((Np, O), F32),
        grid=(gN,),
        in_specs=[
            _rep((Gp, 2 * O + H)),
            _rep((Gp, 1)), _rep((Gp, 1)),
            pl.BlockSpec((TM, 1), lambda j: (j, 0)),
            pl.BlockSpec((TM, D), lambda j: (j, 0)),
            _rep((2 * O + H, H)), _rep((1, H)), _rep((H, O)), _rep((1, O)),
        ],
        out_specs=pl.BlockSpec((TM, O), lambda j: (j, 0)),
        scratch_shapes=[pltpu.VMEM((Gp, O), F32)],
        compiler_params=cp_arb,
    )(pool, cnt_n, cnt_e, ng_c, xg,
      graph_update_w1, _r2(graph_update_b1),
      graph_update_w2, _r2(graph_update_b2))

    return nout[:N], eout[:E], gout[:N]


# speedup vs baseline: 1.4453x; 1.1985x over previous
import functools

import jax
import jax.numpy as jnp
from jax import lax
from jax.experimental import pallas as pl
from jax.experimental.pallas import tpu as pltpu

NEG_SLOPE = 0.01
TM = 512
TME = 512
F32 = jnp.float32
VMEM_LIMIT = 56 * 1024 * 1024


def _lrelu(x):
    return jnp.where(x > 0, x, NEG_SLOPE * x)


def _round_up(n, m):
    return ((n + m - 1) // m) * m


def _pad_rows(x, rows):
    return jnp.pad(x.astype(F32), ((0, rows - x.shape[0]), (0, 0)))


def _pad_idx(idx, rows, fill):
    return jnp.pad(idx.astype(jnp.int32), (0, rows - idx.shape[0]),
                   constant_values=fill)


def _rep(shape):
    return pl.BlockSpec(shape, lambda *_: (0,) * len(shape))


def _r2(b):
    return b.reshape(1, -1)


def _ff_proj_kernel(xn_ref, xg_ref,
                    wn1, bn1, wn2, bn2,
                    wg1, bg1, wg2, bg2,
                    wps, wpdv, wpdu,
                    vu_ref, pp_ref):
    xn = xn_ref[...]
    hv = _lrelu(jnp.dot(xn, wn1[...], preferred_element_type=F32) + bn1[...])
    v = _lrelu(jnp.dot(hv, wn2[...], preferred_element_type=F32) + bn2[...])
    xg = xg_ref[...]
    hu = _lrelu(jnp.dot(xg, wg1[...], preferred_element_type=F32) + bg1[...])
    u = _lrelu(jnp.dot(hu, wg2[...], preferred_element_type=F32) + bg2[...])
    vu_ref[...] = jnp.concatenate([v, u], axis=1)
    ps = jnp.dot(v, wps[...], preferred_element_type=F32)
    pd = (jnp.dot(v, wpdv[...], preferred_element_type=F32)
          + jnp.dot(u, wpdu[...], preferred_element_type=F32))
    pp_ref[...] = jnp.concatenate([ps, pd], axis=1)


def _edge_kernel(n_tiles, n_edges,
                 xe_ref, src_ref, dst_ref, dstp_ref, pp_ref,
                 we1, be1, we2, be2, wee, b1, w2, b2,
                 eout_ref, eagg_ref, gs_scr, gd_scr, pay_scr):
    i = pl.program_id(0)

    @pl.when(i == 0)
    def _():
        nrows = eagg_ref.shape[0]
        zblk = jnp.zeros((128,) + eagg_ref.shape[1:], eagg_ref.dtype)

        def _zero(k, carry):
            eagg_ref[pl.ds(k * 128, 128)] = zblk
            return carry

        lax.fori_loop(0, nrows // 128, _zero, 0)
        pay_scr[...] = jnp.zeros(pay_scr.shape, pay_scr.dtype)

    for mi in range(TME):
        d = dstp_ref[0, 0, mi]
        eagg_ref[d, 0] = eagg_ref[d, 0] + pay_scr[mi]

    xe = xe_ref[...]
    h = _lrelu(jnp.dot(xe, we1[...], preferred_element_type=F32) + be1[...])
    e = _lrelu(jnp.dot(h, we2[...], preferred_element_type=F32) + be2[...])
    q = jnp.dot(e, wee[...], preferred_element_type=F32) + b1[...]

    CH = 8
    for b in range(TME // CH):
        svals = [pp_ref[src_ref[0, 0, b * CH + j], 0] for j in range(CH)]
        dvals = [pp_ref[dst_ref[0, 0, b * CH + j], 0] for j in range(CH)]
        for j in range(CH):
            gs_scr[b * CH + j] = svals[j]
            gd_scr[b * CH + j] = dvals[j]

    hh = gs_scr.shape[1] // 2
    h1 = _lrelu(gs_scr[:, :hh] + gd_scr[:, hh:] + q)
    e_new = _lrelu(jnp.dot(h1, w2[...], preferred_element_type=F32) + b2[...])
    eout_ref[...] = e_new + xe

    base = jnp.minimum(i, n_tiles - 1) * TME
    rows = lax.broadcasted_iota(jnp.int32, (TME, 1), 0) + base
    valid = (rows < n_edges).astype(F32)
    pay_scr[...] = jnp.concatenate([e_new, jnp.ones_like(e_new)], axis=1) * valid


def _node_kernel(vu_ref, eg_ref, xn_ref, ng_ref,
                 w_vu, w_ef, b1, w2, b2,
                 nout_ref, pool_ref):
    j = pl.program_id(0)

    @pl.when(j == 0)
    def _():
        pool_ref[...] = jnp.zeros_like(pool_ref)

    o = nout_ref.shape[1]
    agg = eg_ref[...]
    ef_sum = agg[:, :o]
    deg = agg[:, o:]
    ef = ef_sum * pl.reciprocal(jnp.maximum(deg, 1.0), approx=True)
    vu = vu_ref[...]
    h = _lrelu(jnp.dot(vu, w_vu[...], preferred_element_type=F32)
               + jnp.dot(ef, w_ef[...], preferred_element_type=F32)
               + b1[...])
    n_new = _lrelu(jnp.dot(h, w2[...], preferred_element_type=F32) + b2[...])
    nout_ref[...] = n_new + xn_ref[...]

    gp = pool_ref.shape[0]
    u = vu[:, vu.shape[1] // 2:]
    pooled = jnp.concatenate([n_new, ef_sum, u], axis=1)
    row_ids = lax.broadcasted_iota(jnp.int32, (gp, TM), 0)
    oh = (row_ids == ng_ref[...]).astype(F32)
    pool_ref[...] += jnp.dot(oh, pooled, preferred_element_type=F32)


def _graph_kernel(pool_ref, cntn_ref, cnte_ref, ng_ref, xg_ref,
                  w1, b1, w2, b2,
                  gout_ref, gnew_scr):
    j = pl.program_id(0)
    o = gout_ref.shape[1]

    @pl.when(j == 0)
    def _():
        pool = pool_ref[...]
        gp, width = pool.shape
        inv_n = pl.reciprocal(jnp.maximum(cntn_ref[...], 1.0), approx=True)
        inv_e = pl.reciprocal(jnp.maximum(cnte_ref[...], 1.0), approx=True)
        lane = lax.broadcasted_iota(jnp.int32, (gp, width), 1)
        scale = jnp.where(lane < o, inv_n, jnp.where(lane < 2 * o, inv_e, inv_n))
        cat_g = pool * scale
        hg = _lrelu(jnp.dot(cat_g, w1[...], preferred_element_type=F32) + b1[...])
        gnew_scr[...] = _lrelu(jnp.dot(hg, w2[...], preferred_element_type=F32)
                               + b2[...])

    gp = gnew_scr.shape[0]
    col_ids = lax.broadcasted_iota(jnp.int32, (TM, gp), 1)
    oh = (col_ids == ng_ref[...]).astype(F32)
    gout_ref[...] = (jnp.dot(oh, gnew_scr[...], preferred_element_type=F32)
                     + xg_ref[...])


def kernel(ff_node_w1, ff_node_b1, ff_node_w2, ff_node_b2,
           ff_edge_w1, ff_edge_b1, ff_edge_w2, ff_edge_b2,
           ff_graph_w1, ff_graph_b1, ff_graph_w2, ff_graph_b2,
           edge_update_w1, edge_update_b1, edge_update_w2, edge_update_b2,
           node_update_w1, node_update_b1, node_update_w2, node_update_b2,
           graph_update_w1, graph_update_b1, graph_update_w2, graph_update_b2,
           node_feats, edge_feats, graph_feats,
           src, dst, node_graph, batch_num_nodes, batch_num_edges):
    N, D = node_feats.shape
    E = edge_feats.shape[0]
    G = batch_num_nodes.shape[0]
    H = ff_node_w2.shape[1]
    O = edge_update_w2.shape[1]

    Np = _round_up(max(N, 1), 2 * TM)
    Ep = _round_up(max(E, 1), 2 * TME)
    Gp = _round_up(max(G, 1), 8)
    gN = Np // TM
    gE = Ep // TME

    w_ps = edge_update_w1[0:H]
    w_pdv = edge_update_w1[H:2 * H]
    w_ee = edge_update_w1[2 * H:3 * H]
    w_pdu = edge_update_w1[3 * H:4 * H]
    w_vu = jnp.concatenate([node_update_w1[0:H],
                            node_update_w1[H + O:H + O + H]], axis=0)
    w_ef = node_update_w1[H:H + O]

    xn = _pad_rows(node_feats, Np)
    xg = _pad_rows(graph_feats, Np)
    xe = _pad_rows(edge_feats, Ep)
    src_t = _pad_idx(src, Ep, 0).reshape(Ep // TME, 1, TME)
    dst_t = _pad_idx(dst, Ep, 0).reshape(Ep // TME, 1, TME)
    ng_r = _pad_idx(node_graph, Np, -1).reshape(1, Np)
    ng_c = _pad_idx(node_graph, Np, -1).reshape(Np, 1)
    cnt_n = jnp.zeros((Gp, 1), F32).at[:G, 0].set(batch_num_nodes.astype(F32))
    cnt_e = jnp.zeros((Gp, 1), F32).at[:G, 0].set(batch_num_edges.astype(F32))

    cp_arb = pltpu.CompilerParams(dimension_semantics=("arbitrary",),
                                  vmem_limit_bytes=VMEM_LIMIT)

    vu, pp = pl.pallas_call(
        _ff_proj_kernel,
        out_shape=(jax.ShapeDtypeStruct((Np, 2 * H), F32),
                   jax.ShapeDtypeStruct((Np, 2 * H), F32)),
        grid=(gN,),
        in_specs=[
            pl.BlockSpec((TM, D), lambda i: (i, 0)),
            pl.BlockSpec((TM, D), lambda i: (i, 0)),
            _rep((D, D)), _rep((1, D)), _rep((D, H)), _rep((1, H)),
            _rep((D, D)), _rep((1, D)), _rep((D, H)), _rep((1, H)),
            _rep((H, H)), _rep((H, H)), _rep((H, H)),
        ],
        out_specs=(pl.BlockSpec((TM, 2 * H), lambda i: (i, 0)),
                   pl.BlockSpec((TM, 2 * H), lambda i: (i, 0))),
        compiler_params=cp_arb,
    )(xn, xg,
      ff_node_w1, _r2(ff_node_b1), ff_node_w2, _r2(ff_node_b2),
      ff_graph_w1, _r2(ff_graph_b1), ff_graph_w2, _r2(ff_graph_b2),
      w_ps, w_pdv, w_pdu)

    pp3 = pp.reshape(Np, 1, 2 * H)
    eout, eagg = pl.pallas_call(
        functools.partial(_edge_kernel, gE, E),
        out_shape=(jax.ShapeDtypeStruct((Ep, O), F32),
                   jax.ShapeDtypeStruct((Np, 1, 2 * O), F32)),
        grid=(gE + 1,),
        in_specs=[
            pl.BlockSpec((TME, D),
                         lambda i, g=gE: (jnp.minimum(i, g - 1), 0)),
            pl.BlockSpec((1, 1, TME),
                         lambda i, g=gE: (jnp.minimum(i, g - 1), 0, 0),
                         memory_space=pltpu.SMEM),
            pl.BlockSpec((1, 1, TME),
                         lambda i, g=gE: (jnp.minimum(i, g - 1), 0, 0),
                         memory_space=pltpu.SMEM),
            pl.BlockSpec((1, 1, TME),
                         lambda i: (jnp.maximum(i - 1, 0), 0, 0),
                         memory_space=pltpu.SMEM),
            pl.BlockSpec((Np, 1, 2 * H), lambda i: (0, 0, 0)),
            _rep((D, D)), _rep((1, D)), _rep((D, H)), _rep((1, H)),
            _rep((H, H)), _rep((1, H)), _rep((H, O)), _rep((1, O)),
        ],
        out_specs=(pl.BlockSpec((TME, O),
                                lambda i, g=gE: (jnp.minimum(i, g - 1), 0)),
                   pl.BlockSpec((Np, 1, 2 * O), lambda i: (0, 0, 0))),
        scratch_shapes=[pltpu.VMEM((TME, 2 * H), F32),
                        pltpu.VMEM((TME, 2 * H), F32),
                        pltpu.VMEM((TME, 2 * O), F32)],
        compiler_params=cp_arb,
    )(xe, src_t, dst_t, dst_t, pp3,
      ff_edge_w1, _r2(ff_edge_b1), ff_edge_w2, _r2(ff_edge_b2),
      w_ee, _r2(edge_update_b1), edge_update_w2, _r2(edge_update_b2))

    eagg2 = eagg.reshape(Np, 2 * O)
    nout, pool = pl.pallas_call(
        _node_kernel,
        out_shape=(jax.ShapeDtypeStruct((Np, O), F32),
                   jax.ShapeDtypeStruct((Gp, 2 * O + H), F32)),
        grid=(gN,),
        in_specs=[
            pl.BlockSpec((TM, 2 * H), lambda j: (j, 0)),
            pl.BlockSpec((TM, 2 * O), lambda j: (j, 0)),
            pl.BlockSpec((TM, D), lambda j: (j, 0)),
            pl.BlockSpec((1, TM), lambda j: (0, j)),
            _rep((2 * H, H)), _rep((O, H)), _rep((1, H)),
            _rep((H, O)), _rep((1, O)),
        ],
        out_specs=(pl.BlockSpec((TM, O), lambda j: (j, 0)),
                   pl.BlockSpec((Gp, 2 * O + H), lambda j: (0, 0))),
        compiler_params=cp_arb,
    )(vu, eagg2, xn, ng_r,
      w_vu, w_ef, _r2(node_update_b1), node_update_w2, _r2(node_update_b2))

    gout = pl.pallas_call(
        _graph_kernel,
        out_shape=jax.ShapeDtypeStruct((Np, O), F32),
        grid=(gN,),
        in_specs=[
            _rep((Gp, 2 * O + H)),
            _rep((Gp, 1)), _rep((Gp, 1)),
            pl.BlockSpec((TM, 1), lambda j: (j, 0)),
            pl.BlockSpec((TM, D), lambda j: (j, 0)),
            _rep((2 * O + H, H)), _rep((1, H)), _rep((H, O)), _rep((1, O)),
        ],
        out_specs=pl.BlockSpec((TM, O), lambda j: (j, 0)),
        scratch_shapes=[pltpu.VMEM((Gp, O), F32)],
        compiler_params=cp_arb,
    )(pool, cnt_n, cnt_e, ng_c, xg,
      graph_update_w1, _r2(graph_update_b1),
      graph_update_w2, _r2(graph_update_b2))

    return nout[:N], eout[:E], gout[:N]


# speedup vs baseline: 1.4997x; 1.0376x over previous
import functools

import jax
import jax.numpy as jnp
from jax import lax
from jax.experimental import pallas as pl
from jax.experimental.pallas import tpu as pltpu

NEG_SLOPE = 0.01
TM = 512
TME = 1024
F32 = jnp.float32
VMEM_LIMIT = 56 * 1024 * 1024


def _lrelu(x):
    return jnp.maximum(x, NEG_SLOPE * x)


def _round_up(n, m):
    return ((n + m - 1) // m) * m


def _pad_rows(x, rows):
    x = x.astype(F32)
    if rows == x.shape[0]:
        return x
    return jnp.pad(x, ((0, rows - x.shape[0]), (0, 0)))


def _pad_idx(idx, rows, fill):
    idx = idx.astype(jnp.int32)
    if rows == idx.shape[0]:
        return idx
    return jnp.pad(idx, (0, rows - idx.shape[0]), constant_values=fill)


def _rep(shape):
    return pl.BlockSpec(shape, lambda *_: (0,) * len(shape))


def _r2(b):
    return b.reshape(1, -1)


def _ff_proj_kernel(xn_ref, xg_ref,
                    wn1, bn1, wn2, bn2,
                    wg1, bg1, wg2, bg2,
                    wps, wpdv, wpdu,
                    vu_ref, pp_ref):
    xn = xn_ref[...]
    hv = _lrelu(jnp.dot(xn, wn1[...], preferred_element_type=F32) + bn1[...])
    v = _lrelu(jnp.dot(hv, wn2[...], preferred_element_type=F32) + bn2[...])
    xg = xg_ref[...]
    hu = _lrelu(jnp.dot(xg, wg1[...], preferred_element_type=F32) + bg1[...])
    u = _lrelu(jnp.dot(hu, wg2[...], preferred_element_type=F32) + bg2[...])
    vu_ref[...] = jnp.concatenate([v, u], axis=1)
    ps = jnp.dot(v, wps[...], preferred_element_type=F32)
    pd = (jnp.dot(v, wpdv[...], preferred_element_type=F32)
          + jnp.dot(u, wpdu[...], preferred_element_type=F32))
    pp_ref[...] = jnp.concatenate([ps, pd], axis=1)


def _edge_kernel(n_tiles, n_edges,
                 xe_ref, src_ref, dst_ref, dstp_ref, pp_ref,
                 we1, be1, we2, be2, wee, b1, w2, b2,
                 eout_ref, eagg_ref, gs_scr, gd_scr, pay_scr):
    i = pl.program_id(0)

    @pl.when(i == 0)
    def _():
        nrows = eagg_ref.shape[0]
        zblk = jnp.zeros((128,) + eagg_ref.shape[1:], eagg_ref.dtype)

        def _zero(k, carry):
            eagg_ref[pl.ds(k * 128, 128)] = zblk
            return carry

        lax.fori_loop(0, nrows // 128, _zero, 0)
        pay_scr[...] = jnp.zeros(pay_scr.shape, pay_scr.dtype)

    for mi in range(TME):
        d = dstp_ref[0, 0, mi]
        eagg_ref[d, 0] = eagg_ref[d, 0] + pay_scr[mi]

    xe = xe_ref[...]
    h = _lrelu(jnp.dot(xe, we1[...], preferred_element_type=F32) + be1[...])
    e = _lrelu(jnp.dot(h, we2[...], preferred_element_type=F32) + be2[...])
    q = jnp.dot(e, wee[...], preferred_element_type=F32) + b1[...]

    CH = 8
    for b in range(TME // CH):
        svals = [pp_ref[src_ref[0, 0, b * CH + j], 0] for j in range(CH)]
        dvals = [pp_ref[dst_ref[0, 0, b * CH + j], 0] for j in range(CH)]
        for j in range(CH):
            gs_scr[b * CH + j] = svals[j]
            gd_scr[b * CH + j] = dvals[j]

    hh = gs_scr.shape[1] // 2
    h1 = _lrelu(gs_scr[:, :hh] + gd_scr[:, hh:] + q)
    e_new = _lrelu(jnp.dot(h1, w2[...], preferred_element_type=F32) + b2[...])
    eout_ref[...] = e_new + xe

    base = jnp.minimum(i, n_tiles - 1) * TME
    rows = lax.broadcasted_iota(jnp.int32, (TME, 1), 0) + base
    valid = (rows < n_edges).astype(F32)
    pay_scr[...] = jnp.concatenate([e_new, jnp.ones_like(e_new)], axis=1) * valid


def _node_kernel(vu_ref, eg_ref, xn_ref, ng_ref,
                 w_vu, w_ef, b1, w2, b2,
                 nout_ref, pool_ref):
    j = pl.program_id(0)

    @pl.when(j == 0)
    def _():
        pool_ref[...] = jnp.zeros_like(pool_ref)

    o = nout_ref.shape[1]
    agg = eg_ref[...]
    ef_sum = agg[:, :o]
    deg = agg[:, o:]
    ef = ef_sum * pl.reciprocal(jnp.maximum(deg, 1.0), approx=True)
    vu = vu_ref[...]
    h = _lrelu(jnp.dot(vu, w_vu[...], preferred_element_type=F32)
               + jnp.dot(ef, w_ef[...], preferred_element_type=F32)
               + b1[...])
    n_new = _lrelu(jnp.dot(h, w2[...], preferred_element_type=F32) + b2[...])
    nout_ref[...] = n_new + xn_ref[...]

    gp = pool_ref.shape[0]
    u = vu[:, vu.shape[1] // 2:]
    pooled = jnp.concatenate([n_new, ef_sum, u], axis=1)
    row_ids = lax.broadcasted_iota(jnp.int32, (gp, TM), 0)
    oh = (row_ids == ng_ref[...]).astype(F32)
    pool_ref[...] += jnp.dot(oh, pooled, preferred_element_type=F32)


def _graph_kernel(pool_ref, cntn_ref, cnte_ref, ng_ref, xg_ref,
                  w1, b1, w2, b2,
                  gout_ref, gnew_scr):
    j = pl.program_id(0)
    o = gout_ref.shape[1]

    @pl.when(j == 0)
    def _():
        pool = pool_ref[...]
        gp, width = pool.shape
        inv_n = pl.reciprocal(jnp.maximum(cntn_ref[...], 1.0), approx=True)
        inv_e = pl.reciprocal(jnp.maximum(cnte_ref[...], 1.0), approx=True)
        lane = lax.broadcasted_iota(jnp.int32, (gp, width), 1)
        scale = jnp.where(lane < o, inv_n, jnp.where(lane < 2 * o, inv_e, inv_n))
        cat_g = pool * scale
        hg = _lrelu(jnp.dot(cat_g, w1[...], preferred_element_type=F32) + b1[...])
        gnew_scr[...] = _lrelu(jnp.dot(hg, w2[...], preferred_element_type=F32)
                               + b2[...])

    gp = gnew_scr.shape[0]
    col_ids = lax.broadcasted_iota(jnp.int32, (TM, gp), 1)
    oh = (col_ids == ng_ref[...]).astype(F32)
    gout_ref[...] = (jnp.dot(oh, gnew_scr[...], preferred_element_type=F32)
                     + xg_ref[...])


def kernel(ff_node_w1, ff_node_b1, ff_node_w2, ff_node_b2,
           ff_edge_w1, ff_edge_b1, ff_edge_w2, ff_edge_b2,
           ff_graph_w1, ff_graph_b1, ff_graph_w2, ff_graph_b2,
           edge_update_w1, edge_update_b1, edge_update_w2, edge_update_b2,
           node_update_w1, node_update_b1, node_update_w2, node_update_b2,
           graph_update_w1, graph_update_b1, graph_update_w2, graph_update_b2,
           node_feats, edge_feats, graph_feats,
           src, dst, node_graph, batch_num_nodes, batch_num_edges):
    N, D = node_feats.shape
    E = edge_feats.shape[0]
    G = batch_num_nodes.shape[0]
    H = ff_node_w2.shape[1]
    O = edge_update_w2.shape[1]

    Np = _round_up(max(N, 1), 2 * TM)
    Ep = _round_up(max(E, 1), 2 * TME)
    Gp = _round_up(max(G, 1), 8)
    gN = Np // TM
    gE = Ep // TME

    w_ps = edge_update_w1[0:H]
    w_pdv = edge_update_w1[H:2 * H]
    w_ee = edge_update_w1[2 * H:3 * H]
    w_pdu = edge_update_w1[3 * H:4 * H]
    w_vu = jnp.concatenate([node_update_w1[0:H],
                            node_update_w1[H + O:H + O + H]], axis=0)
    w_ef = node_update_w1[H:H + O]

    xn = _pad_rows(node_feats, Np)
    xg = _pad_rows(graph_feats, Np)
    xe = _pad_rows(edge_feats, Ep)
    src_t = _pad_idx(src, Ep, 0).reshape(Ep // TME, 1, TME)
    dst_t = _pad_idx(dst, Ep, 0).reshape(Ep // TME, 1, TME)
    ng_r = _pad_idx(node_graph, Np, -1).reshape(1, Np)
    ng_c = _pad_idx(node_graph, Np, -1).reshape(Np, 1)
    cnt_n = jnp.zeros((Gp, 1), F32).at[:G, 0].set(batch_num_nodes.astype(F32))
    cnt_e = jnp.zeros((Gp, 1), F32).at[:G, 0].set(batch_num_edges.astype(F32))

    cp_arb = pltpu.CompilerParams(dimension_semantics=("arbitrary",),
                                  vmem_limit_bytes=VMEM_LIMIT)

    vu, pp = pl.pallas_call(
        _ff_proj_kernel,
        out_shape=(jax.ShapeDtypeStruct((Np, 2 * H), F32),
                   jax.ShapeDtypeStruct((Np, 2 * H), F32)),
        grid=(gN,),
        in_specs=[
            pl.BlockSpec((TM, D), lambda i: (i, 0)),
            pl.BlockSpec((TM, D), lambda i: (i, 0)),
            _rep((D, D)), _rep((1, D)), _rep((D, H)), _rep((1, H)),
            _rep((D, D)), _rep((1, D)), _rep((D, H)), _rep((1, H)),
            _rep((H, H)), _rep((H, H)), _rep((H, H)),
        ],
        out_specs=(pl.BlockSpec((TM, 2 * H), lambda i: (i, 0)),
                   pl.BlockSpec((TM, 2 * H), lambda i: (i, 0))),
        compiler_params=cp_arb,
    )(xn, xg,
      ff_node_w1, _r2(ff_node_b1), ff_node_w2, _r2(ff_node_b2),
      ff_graph_w1, _r2(ff_graph_b1), ff_graph_w2, _r2(ff_graph_b2),
      w_ps, w_pdv, w_pdu)

    pp3 = pp.reshape(Np, 1, 2 * H)
    eout, eagg = pl.pallas_call(
        functools.partial(_edge_kernel, gE, E),
        out_shape=(jax.ShapeDtypeStruct((Ep, O), F32),
                   jax.ShapeDtypeStruct((Np, 1, 2 * O), F32)),
        grid=(gE + 1,),
        in_specs=[
            pl.BlockSpec((TME, D),
                         lambda i, g=gE: (jnp.minimum(i, g - 1), 0)),
            pl.BlockSpec((1, 1, TME),
                         lambda i, g=gE: (jnp.minimum(i, g - 1), 0, 0),
                         memory_space=pltpu.SMEM),
            pl.BlockSpec((1, 1, TME),
                         lambda i, g=gE: (jnp.minimum(i, g - 1), 0, 0),
                         memory_space=pltpu.SMEM),
            pl.BlockSpec((1, 1, TME),
                         lambda i: (jnp.maximum(i - 1, 0), 0, 0),
                         memory_space=pltpu.SMEM),
            pl.BlockSpec((Np, 1, 2 * H), lambda i: (0, 0, 0)),
            _rep((D, D)), _rep((1, D)), _rep((D, H)), _rep((1, H)),
            _rep((H, H)), _rep((1, H)), _rep((H, O)), _rep((1, O)),
        ],
        out_specs=(pl.BlockSpec((TME, O),
                                lambda i, g=gE: (jnp.minimum(i, g - 1), 0)),
                   pl.BlockSpec((Np, 1, 2 * O), lambda i: (0, 0, 0))),
        scratch_shapes=[pltpu.VMEM((TME, 2 * H), F32),
                        pltpu.VMEM((TME, 2 * H), F32),
                        pltpu.VMEM((TME, 2 * O), F32)],
        compiler_params=cp_arb,
    )(xe, src_t, dst_t, dst_t, pp3,
      ff_edge_w1, _r2(ff_edge_b1), ff_edge_w2, _r2(ff_edge_b2),
      w_ee, _r2(edge_update_b1), edge_update_w2, _r2(edge_update_b2))

    eagg2 = eagg.reshape(Np, 2 * O)
    nout, pool = pl.pallas_call(
        _node_kernel,
        out_shape=(jax.ShapeDtypeStruct((Np, O), F32),
                   jax.ShapeDtypeStruct((Gp, 2 * O + H), F32)),
        grid=(gN,),
        in_specs=[
            pl.BlockSpec((TM, 2 * H), lambda j: (j, 0)),
            pl.BlockSpec((TM, 2 * O), lambda j: (j, 0)),
            pl.BlockSpec((TM, D), lambda j: (j, 0)),
            pl.BlockSpec((1, TM), lambda j: (0, j)),
            _rep((2 * H, H)), _rep((O, H)), _rep((1, H)),
            _rep((H, O)), _rep((1, O)),
        ],
        out_specs=(pl.BlockSpec((TM, O), lambda j: (j, 0)),
                   pl.BlockSpec((Gp, 2 * O + H), lambda j: (0, 0))),
        compiler_params=cp_arb,
    )(vu, eagg2, xn, ng_r,
      w_vu, w_ef, _r2(node_update_b1), node_update_w2, _r2(node_update_b2))

    gout = pl.pallas_call(
        _graph_kernel,
        out_shape=jax.ShapeDtypeStruct((Np, O), F32),
        grid=(gN,),
        in_specs=[
            _rep((Gp, 2 * O + H)),
            _rep((Gp, 1)), _rep((Gp, 1)),
            pl.BlockSpec((TM, 1), lambda j: (j, 0)),
            pl.BlockSpec((TM, D), lambda j: (j, 0)),
            _rep((2 * O + H, H)), _rep((1, H)), _rep((H, O)), _rep((1, O)),
        ],
        out_specs=pl.BlockSpec((TM, O), lambda j: (j, 0)),
        scratch_shapes=[pltpu.VMEM((Gp, O), F32)],
        compiler_params=cp_arb,
    )(pool, cnt_n, cnt_e, ng_c, xg,
      graph_update_w1, _r2(graph_update_b1),
      graph_update_w2, _r2(graph_update_b2))

    return nout[:N], eout[:E], gout[:N]


# speedup vs baseline: 1.5467x; 1.0314x over previous
import functools

import jax
import jax.numpy as jnp
from jax import lax
from jax.experimental import pallas as pl
from jax.experimental.pallas import tpu as pltpu

NEG_SLOPE = 0.01
TM = 512
TME = 1024
F32 = jnp.float32
VMEM_LIMIT = 56 * 1024 * 1024


def _lrelu(x):
    return jnp.maximum(x, NEG_SLOPE * x)


def _round_up(n, m):
    return ((n + m - 1) // m) * m


def _pad_rows(x, rows):
    x = x.astype(F32)
    if rows == x.shape[0]:
        return x
    return jnp.pad(x, ((0, rows - x.shape[0]), (0, 0)))


def _pad_idx(idx, rows, fill):
    idx = idx.astype(jnp.int32)
    if rows == idx.shape[0]:
        return idx
    return jnp.pad(idx, (0, rows - idx.shape[0]), constant_values=fill)


def _rep(shape):
    return pl.BlockSpec(shape, lambda *_: (0,) * len(shape))


def _r2(b):
    return b.reshape(1, -1)


def _ff_proj_kernel(xn_ref, xg_ref,
                    wn1, bn1, wn2, bn2,
                    wg1, bg1, wg2, bg2,
                    wps, wpdv, wpdu,
                    vu_ref, pp_ref):
    xn = xn_ref[...]
    hv = _lrelu(jnp.dot(xn, wn1[...], preferred_element_type=F32) + bn1[...])
    v = _lrelu(jnp.dot(hv, wn2[...], preferred_element_type=F32) + bn2[...])
    xg = xg_ref[...]
    hu = _lrelu(jnp.dot(xg, wg1[...], preferred_element_type=F32) + bg1[...])
    u = _lrelu(jnp.dot(hu, wg2[...], preferred_element_type=F32) + bg2[...])
    vu_ref[...] = jnp.concatenate([v, u], axis=1)
    ps = jnp.dot(v, wps[...], preferred_element_type=F32)
    pd = (jnp.dot(v, wpdv[...], preferred_element_type=F32)
          + jnp.dot(u, wpdu[...], preferred_element_type=F32))
    pp_ref[...] = jnp.concatenate([ps, pd], axis=1)


def _edge_kernel(n_tiles, n_edges,
                 xe_ref, src_ref, dst_ref, dstp_ref, pp_ref,
                 we1, be1, we2, be2, wee, b1, w2, b2,
                 eout_ref, eagg_ref, gs_scr, gd_scr, pay_scr):
    i = pl.program_id(0)

    @pl.when(i == 0)
    def _():
        nrows = eagg_ref.shape[0]
        zblk = jnp.zeros((128,) + eagg_ref.shape[1:], eagg_ref.dtype)

        def _zero(k, carry):
            eagg_ref[pl.ds(k * 128, 128)] = zblk
            return carry

        lax.fori_loop(0, nrows // 128, _zero, 0)
        pay_scr[...] = jnp.zeros(pay_scr.shape, pay_scr.dtype)

    xe = xe_ref[...]
    h = _lrelu(jnp.dot(xe, we1[...], preferred_element_type=F32) + be1[...])
    e = _lrelu(jnp.dot(h, we2[...], preferred_element_type=F32) + be2[...])
    q = jnp.dot(e, wee[...], preferred_element_type=F32) + b1[...]

    CH = 8
    for b in range(TME // CH):
        svals = [pp_ref[src_ref[0, 0, b * CH + j], 0] for j in range(CH)]
        dvals = [pp_ref[dst_ref[0, 0, b * CH + j], 0] for j in range(CH)]
        for j in range(CH):
            gs_scr[b * CH + j] = svals[j]
            gd_scr[b * CH + j] = dvals[j]
        for j in range(CH):
            mi = b * CH + j
            d = dstp_ref[0, 0, mi]
            eagg_ref[d, 0] = eagg_ref[d, 0] + pay_scr[mi]

    hh = gs_scr.shape[1] // 2
    h1 = _lrelu(gs_scr[:, :hh] + gd_scr[:, hh:] + q)
    e_new = _lrelu(jnp.dot(h1, w2[...], preferred_element_type=F32) + b2[...])
    eout_ref[...] = e_new + xe

    base = jnp.minimum(i, n_tiles - 1) * TME
    rows = lax.broadcasted_iota(jnp.int32, (TME, 1), 0) + base
    valid = (rows < n_edges).astype(F32)
    pay_scr[...] = jnp.concatenate([e_new, jnp.ones_like(e_new)], axis=1) * valid


def _node_kernel(vu_ref, eg_ref, xn_ref, ng_ref,
                 w_vu, w_ef, b1, w2, b2,
                 nout_ref, pool_ref):
    j = pl.program_id(0)

    @pl.when(j == 0)
    def _():
        pool_ref[...] = jnp.zeros_like(pool_ref)

    o = nout_ref.shape[1]
    agg = eg_ref[...]
    ef_sum = agg[:, :o]
    deg = agg[:, o:]
    ef = ef_sum * pl.reciprocal(jnp.maximum(deg, 1.0), approx=True)
    vu = vu_ref[...]
    h = _lrelu(jnp.dot(vu, w_vu[...], preferred_element_type=F32)
               + jnp.dot(ef, w_ef[...], preferred_element_type=F32)
               + b1[...])
    n_new = _lrelu(jnp.dot(h, w2[...], preferred_element_type=F32) + b2[...])
    nout_ref[...] = n_new + xn_ref[...]

    gp = pool_ref.shape[0]
    u = vu[:, vu.shape[1] // 2:]
    pooled = jnp.concatenate([n_new, ef_sum, u], axis=1)
    row_ids = lax.broadcasted_iota(jnp.int32, (gp, TM), 0)
    oh = (row_ids == ng_ref[...]).astype(F32)
    pool_ref[...] += jnp.dot(oh, pooled, preferred_element_type=F32)


def _graph_kernel(pool_ref, cntn_ref, cnte_ref, ng_ref, xg_ref,
                  w1, b1, w2, b2,
                  gout_ref, gnew_scr):
    j = pl.program_id(0)
    o = gout_ref.shape[1]

    @pl.when(j == 0)
    def _():
        pool = pool_ref[...]
        gp, width = pool.shape
        inv_n = pl.reciprocal(jnp.maximum(cntn_ref[...], 1.0), approx=True)
        inv_e = pl.reciprocal(jnp.maximum(cnte_ref[...], 1.0), approx=True)
        lane = lax.broadcasted_iota(jnp.int32, (gp, width), 1)
        scale = jnp.where(lane < o, inv_n, jnp.where(lane < 2 * o, inv_e, inv_n))
        cat_g = pool * scale
        hg = _lrelu(jnp.dot(cat_g, w1[...], preferred_element_type=F32) + b1[...])
        gnew_scr[...] = _lrelu(jnp.dot(hg, w2[...], preferred_element_type=F32)
                               + b2[...])

    gp = gnew_scr.shape[0]
    col_ids = lax.broadcasted_iota(jnp.int32, (TM, gp), 1)
    oh = (col_ids == ng_ref[...]).astype(F32)
    gout_ref[...] = (jnp.dot(oh, gnew_scr[...], preferred_element_type=F32)
                     + xg_ref[...])


def kernel(ff_node_w1, ff_node_b1, ff_node_w2, ff_node_b2,
           ff_edge_w1, ff_edge_b1, ff_edge_w2, ff_edge_b2,
           ff_graph_w1, ff_graph_b1, ff_graph_w2, ff_graph_b2,
           edge_update_w1, edge_update_b1, edge_update_w2, edge_update_b2,
           node_update_w1, node_update_b1, node_update_w2, node_update_b2,
           graph_update_w1, graph_update_b1, graph_update_w2, graph_update_b2,
           node_feats, edge_feats, graph_feats,
           src, dst, node_graph, batch_num_nodes, batch_num_edges):
    N, D = node_feats.shape
    E = edge_feats.shape[0]
    G = batch_num_nodes.shape[0]
    H = ff_node_w2.shape[1]
    O = edge_update_w2.shape[1]

    Np = _round_up(max(N, 1), 2 * TM)
    Ep = _round_up(max(E, 1), 2 * TME)
    Gp = _round_up(max(G, 1), 8)
    gN = Np // TM
    gE = Ep // TME

    w_ps = edge_update_w1[0:H]
    w_pdv = edge_update_w1[H:2 * H]
    w_ee = edge_update_w1[2 * H:3 * H]
    w_pdu = edge_update_w1[3 * H:4 * H]
    w_vu = jnp.concatenate([node_update_w1[0:H],
                            node_update_w1[H + O:H + O + H]], axis=0)
    w_ef = node_update_w1[H:H + O]

    xn = _pad_rows(node_feats, Np)
    xg = _pad_rows(graph_feats, Np)
    xe = _pad_rows(edge_feats, Ep)
    src_t = _pad_idx(src, Ep, 0).reshape(Ep // TME, 1, TME)
    dst_t = _pad_idx(dst, Ep, 0).reshape(Ep // TME, 1, TME)
    ng_r = _pad_idx(node_graph, Np, -1).reshape(1, Np)
    ng_c = _pad_idx(node_graph, Np, -1).reshape(Np, 1)
    cnt_n = jnp.zeros((Gp, 1), F32).at[:G, 0].set(batch_num_nodes.astype(F32))
    cnt_e = jnp.zeros((Gp, 1), F32).at[:G, 0].set(batch_num_edges.astype(F32))

    cp_arb = pltpu.CompilerParams(dimension_semantics=("arbitrary",),
                                  vmem_limit_bytes=VMEM_LIMIT)

    vu, pp = pl.pallas_call(
        _ff_proj_kernel,
        out_shape=(jax.ShapeDtypeStruct((Np, 2 * H), F32),
                   jax.ShapeDtypeStruct((Np, 2 * H), F32)),
        grid=(gN,),
        in_specs=[
            pl.BlockSpec((TM, D), lambda i: (i, 0)),
            pl.BlockSpec((TM, D), lambda i: (i, 0)),
            _rep((D, D)), _rep((1, D)), _rep((D, H)), _rep((1, H)),
            _rep((D, D)), _rep((1, D)), _rep((D, H)), _rep((1, H)),
            _rep((H, H)), _rep((H, H)), _rep((H, H)),
        ],
        out_specs=(pl.BlockSpec((TM, 2 * H), lambda i: (i, 0)),
                   pl.BlockSpec((TM, 2 * H), lambda i: (i, 0))),
        compiler_params=cp_arb,
    )(xn, xg,
      ff_node_w1, _r2(ff_node_b1), ff_node_w2, _r2(ff_node_b2),
      ff_graph_w1, _r2(ff_graph_b1), ff_graph_w2, _r2(ff_graph_b2),
      w_ps, w_pdv, w_pdu)

    pp3 = pp.reshape(Np, 1, 2 * H)
    eout, eagg = pl.pallas_call(
        functools.partial(_edge_kernel, gE, E),
        out_shape=(jax.ShapeDtypeStruct((Ep, O), F32),
                   jax.ShapeDtypeStruct((Np, 1, 2 * O), F32)),
        grid=(gE + 1,),
        in_specs=[
            pl.BlockSpec((TME, D),
                         lambda i, g=gE: (jnp.minimum(i, g - 1), 0)),
            pl.BlockSpec((1, 1, TME),
                         lambda i, g=gE: (jnp.minimum(i, g - 1), 0, 0),
                         memory_space=pltpu.SMEM),
            pl.BlockSpec((1, 1, TME),
                         lambda i, g=gE: (jnp.minimum(i, g - 1), 0, 0),
                         memory_space=pltpu.SMEM),
            pl.BlockSpec((1, 1, TME),
                         lambda i: (jnp.maximum(i - 1, 0), 0, 0),
                         memory_space=pltpu.SMEM),
            pl.BlockSpec((Np, 1, 2 * H), lambda i: (0, 0, 0)),
            _rep((D, D)), _rep((1, D)), _rep((D, H)), _rep((1, H)),
            _rep((H, H)), _rep((1, H)), _rep((H, O)), _rep((1, O)),
        ],
        out_specs=(pl.BlockSpec((TME, O),
                                lambda i, g=gE: (jnp.minimum(i, g - 1), 0)),
                   pl.BlockSpec((Np, 1, 2 * O), lambda i: (0, 0, 0))),
        scratch_shapes=[pltpu.VMEM((TME, 2 * H), F32),
                        pltpu.VMEM((TME, 2 * H), F32),
                        pltpu.VMEM((TME, 2 * O), F32)],
        compiler_params=cp_arb,
    )(xe, src_t, dst_t, dst_t, pp3,
      ff_edge_w1, _r2(ff_edge_b1), ff_edge_w2, _r2(ff_edge_b2),
      w_ee, _r2(edge_update_b1), edge_update_w2, _r2(edge_update_b2))

    eagg2 = eagg.reshape(Np, 2 * O)
    nout, pool = pl.pallas_call(
        _node_kernel,
        out_shape=(jax.ShapeDtypeStruct((Np, O), F32),
                   jax.ShapeDtypeStruct((Gp, 2 * O + H), F32)),
        grid=(gN,),
        in_specs=[
            pl.BlockSpec((TM, 2 * H), lambda j: (j, 0)),
            pl.BlockSpec((TM, 2 * O), lambda j: (j, 0)),
            pl.BlockSpec((TM, D), lambda j: (j, 0)),
            pl.BlockSpec((1, TM), lambda j: (0, j)),
            _rep((2 * H, H)), _rep((O, H)), _rep((1, H)),
            _rep((H, O)), _rep((1, O)),
        ],
        out_specs=(pl.BlockSpec((TM, O), lambda j: (j, 0)),
                   pl.BlockSpec((Gp, 2 * O + H), lambda j: (0, 0))),
        compiler_params=cp_arb,
    )(vu, eagg2, xn, ng_r,
      w_vu, w_ef, _r2(node_update_b1), node_update_w2, _r2(node_update_b2))

    gout = pl.pallas_call(
        _graph_kernel,
        out_shape=jax.ShapeDtypeStruct((Np, O), F32),
        grid=(gN,),
        in_specs=[
            _rep((Gp, 2 * O + H)),
            _rep((Gp, 1)), _rep((Gp, 1)),
            pl.BlockSpec((TM, 1), lambda j: (j, 0)),
            pl.BlockSpec((TM, D), lambda j: (j, 0)),
            _rep((2 * O + H, H)), _rep((1, H)), _rep((H, O)), _rep((1, O)),
        ],
        out_specs=pl.BlockSpec((TM, O), lambda j: (j, 0)),
        scratch_shapes=[pltpu.VMEM((Gp, O), F32)],
        compiler_params=cp_arb,
    )(pool, cnt_n, cnt_e, ng_c, xg,
      graph_update_w1, _r2(graph_update_b1),
      graph_update_w2, _r2(graph_update_b2))

    return nout[:N], eout[:E], gout[:N]


# speedup vs baseline: 1.9791x; 1.2796x over previous
import functools

import jax
import jax.numpy as jnp
from jax import lax
from jax.experimental import pallas as pl
from jax.experimental.pallas import tpu as pltpu

NEG_SLOPE = 0.01
TM = 512
TME = 1024
F32 = jnp.float32
VMEM_LIMIT = 56 * 1024 * 1024


def _lrelu(x):
    return jnp.maximum(x, NEG_SLOPE * x)


def _round_up(n, m):
    return ((n + m - 1) // m) * m


def _pad_rows(x, rows):
    x = x.astype(F32)
    if rows == x.shape[0]:
        return x
    return jnp.pad(x, ((0, rows - x.shape[0]), (0, 0)))


def _pad_idx(idx, rows, fill):
    idx = idx.astype(jnp.int32)
    if rows == idx.shape[0]:
        return idx
    return jnp.pad(idx, (0, rows - idx.shape[0]), constant_values=fill)


def _rep(shape):
    return pl.BlockSpec(shape, lambda *_: (0,) * len(shape))


def _r2(b):
    return b.reshape(1, -1)


def _ff_proj_kernel(xn_ref, xg_ref,
                    wn1, bn1, wn2, bn2,
                    wg1, bg1, wg2, bg2,
                    wps, wpdv, wpdu,
                    vu_ref, pp_ref):
    xn = xn_ref[...]
    hv = _lrelu(jnp.dot(xn, wn1[...], preferred_element_type=F32) + bn1[...])
    v = _lrelu(jnp.dot(hv, wn2[...], preferred_element_type=F32) + bn2[...])
    xg = xg_ref[...]
    hu = _lrelu(jnp.dot(xg, wg1[...], preferred_element_type=F32) + bg1[...])
    u = _lrelu(jnp.dot(hu, wg2[...], preferred_element_type=F32) + bg2[...])
    vu_ref[...] = jnp.concatenate([v, u], axis=1)
    ps = jnp.dot(v, wps[...], preferred_element_type=F32)
    pd = (jnp.dot(v, wpdv[...], preferred_element_type=F32)
          + jnp.dot(u, wpdu[...], preferred_element_type=F32))
    pp_ref[...] = jnp.concatenate([ps, pd], axis=1)


def _edge_kernel(n_tiles, n_edges,
                 xe_ref, src_ref, dst_ref, dstp_ref, pp_ref,
                 we1, be1, we2, be2, wee, b1, w2, b2,
                 eout_ref, ega_ref, egb_ref, gs_scr, gd_scr, pay_scr):
    i = pl.program_id(0)

    @pl.when(i == 0)
    def _():
        nrows = ega_ref.shape[0]
        zblk = jnp.zeros((128,) + ega_ref.shape[1:], ega_ref.dtype)

        def _zero(k, carry):
            ega_ref[pl.ds(k * 128, 128)] = zblk
            egb_ref[pl.ds(k * 128, 128)] = zblk
            return carry

        lax.fori_loop(0, nrows // 128, _zero, 0)
        pay_scr[...] = jnp.zeros(pay_scr.shape, pay_scr.dtype)

    xe = xe_ref[...]
    h = _lrelu(jnp.dot(xe, we1[...], preferred_element_type=F32) + be1[...])
    e = _lrelu(jnp.dot(h, we2[...], preferred_element_type=F32) + be2[...])
    q = jnp.dot(e, wee[...], preferred_element_type=F32) + b1[...]

    CH = 8
    for b in range(TME // CH):
        svals = [pp_ref[src_ref[0, 0, b * CH + j], 0] for j in range(CH)]
        dvals = [pp_ref[dst_ref[0, 0, b * CH + j], 0] for j in range(CH)]
        for j in range(CH):
            gs_scr[b * CH + j] = svals[j]
            gd_scr[b * CH + j] = dvals[j]
        for j in range(CH):
            mi = b * CH + j
            d = dstp_ref[0, 0, mi]
            acc = ega_ref if j % 2 == 0 else egb_ref
            acc[d, 0] = acc[d, 0] + pay_scr[mi]

    hh = gs_scr.shape[1] // 2
    h1 = _lrelu(gs_scr[:, :hh] + gd_scr[:, hh:] + q)
    e_new = _lrelu(jnp.dot(h1, w2[...], preferred_element_type=F32) + b2[...])
    eout_ref[...] = e_new + xe

    base = jnp.minimum(i, n_tiles - 1) * TME
    rows = lax.broadcasted_iota(jnp.int32, (TME, 1), 0) + base
    valid = (rows < n_edges).astype(F32)
    pay_scr[...] = jnp.concatenate([e_new, jnp.ones_like(e_new)], axis=1) * valid


def _node_kernel(vu_ref, ega_ref, egb_ref, xn_ref, ng_ref,
                 w_vu, w_ef, b1, w2, b2,
                 nout_ref, pool_ref):
    j = pl.program_id(0)

    @pl.when(j == 0)
    def _():
        pool_ref[...] = jnp.zeros_like(pool_ref)

    o = nout_ref.shape[1]
    agg = ega_ref[...] + egb_ref[...]
    ef_sum = agg[:, :o]
    deg = agg[:, o:]
    ef = ef_sum * pl.reciprocal(jnp.maximum(deg, 1.0), approx=True)
    vu = vu_ref[...]
    h = _lrelu(jnp.dot(vu, w_vu[...], preferred_element_type=F32)
               + jnp.dot(ef, w_ef[...], preferred_element_type=F32)
               + b1[...])
    n_new = _lrelu(jnp.dot(h, w2[...], preferred_element_type=F32) + b2[...])
    nout_ref[...] = n_new + xn_ref[...]

    gp = pool_ref.shape[0]
    u = vu[:, vu.shape[1] // 2:]
    pooled = jnp.concatenate([n_new, ef_sum, u], axis=1)
    row_ids = lax.broadcasted_iota(jnp.int32, (gp, TM), 0)
    oh = (row_ids == ng_ref[...]).astype(F32)
    pool_ref[...] += jnp.dot(oh, pooled, preferred_element_type=F32)


def _graph_kernel(pool_ref, cntn_ref, cnte_ref, ng_ref, xg_ref,
                  w1, b1, w2, b2,
                  gout_ref, gnew_scr):
    j = pl.program_id(0)
    o = gout_ref.shape[1]

    @pl.when(j == 0)
    def _():
        pool = pool_ref[...]
        gp, width = pool.shape
        inv_n = pl.reciprocal(jnp.maximum(cntn_ref[...], 1.0), approx=True)
        inv_e = pl.reciprocal(jnp.maximum(cnte_ref[...], 1.0), approx=True)
        lane = lax.broadcasted_iota(jnp.int32, (gp, width), 1)
        scale = jnp.where(lane < o, inv_n, jnp.where(lane < 2 * o, inv_e, inv_n))
        cat_g = pool * scale
        hg = _lrelu(jnp.dot(cat_g, w1[...], preferred_element_type=F32) + b1[...])
        gnew_scr[...] = _lrelu(jnp.dot(hg, w2[...], preferred_element_type=F32)
                               + b2[...])

    gp = gnew_scr.shape[0]
    col_ids = lax.broadcasted_iota(jnp.int32, (TM, gp), 1)
    oh = (col_ids == ng_ref[...]).astype(F32)
    gout_ref[...] = (jnp.dot(oh, gnew_scr[...], preferred_element_type=F32)
                     + xg_ref[...])


def kernel(ff_node_w1, ff_node_b1, ff_node_w2, ff_node_b2,
           ff_edge_w1, ff_edge_b1, ff_edge_w2, ff_edge_b2,
           ff_graph_w1, ff_graph_b1, ff_graph_w2, ff_graph_b2,
           edge_update_w1, edge_update_b1, edge_update_w2, edge_update_b2,
           node_update_w1, node_update_b1, node_update_w2, node_update_b2,
           graph_update_w1, graph_update_b1, graph_update_w2, graph_update_b2,
           node_feats, edge_feats, graph_feats,
           src, dst, node_graph, batch_num_nodes, batch_num_edges):
    N, D = node_feats.shape
    E = edge_feats.shape[0]
    G = batch_num_nodes.shape[0]
    H = ff_node_w2.shape[1]
    O = edge_update_w2.shape[1]

    Np = _round_up(max(N, 1), 2 * TM)
    Ep = _round_up(max(E, 1), 2 * TME)
    Gp = _round_up(max(G, 1), 8)
    gN = Np // TM
    gE = Ep // TME

    w_ps = edge_update_w1[0:H]
    w_pdv = edge_update_w1[H:2 * H]
    w_ee = edge_update_w1[2 * H:3 * H]
    w_pdu = edge_update_w1[3 * H:4 * H]
    w_vu = jnp.concatenate([node_update_w1[0:H],
                            node_update_w1[H + O:H + O + H]], axis=0)
    w_ef = node_update_w1[H:H + O]

    xn = _pad_rows(node_feats, Np)
    xg = _pad_rows(graph_feats, Np)
    xe = _pad_rows(edge_feats, Ep)
    src_t = _pad_idx(src, Ep, 0).reshape(Ep // TME, 1, TME)
    dst_t = _pad_idx(dst, Ep, 0).reshape(Ep // TME, 1, TME)
    ng_r = _pad_idx(node_graph, Np, -1).reshape(1, Np)
    ng_c = _pad_idx(node_graph, Np, -1).reshape(Np, 1)
    cnt_n = jnp.zeros((Gp, 1), F32).at[:G, 0].set(batch_num_nodes.astype(F32))
    cnt_e = jnp.zeros((Gp, 1), F32).at[:G, 0].set(batch_num_edges.astype(F32))

    cp_arb = pltpu.CompilerParams(dimension_semantics=("arbitrary",),
                                  vmem_limit_bytes=VMEM_LIMIT)

    vu, pp = pl.pallas_call(
        _ff_proj_kernel,
        out_shape=(jax.ShapeDtypeStruct((Np, 2 * H), F32),
                   jax.ShapeDtypeStruct((Np, 2 * H), F32)),
        grid=(gN,),
        in_specs=[
            pl.BlockSpec((TM, D), lambda i: (i, 0)),
            pl.BlockSpec((TM, D), lambda i: (i, 0)),
            _rep((D, D)), _rep((1, D)), _rep((D, H)), _rep((1, H)),
            _rep((D, D)), _rep((1, D)), _rep((D, H)), _rep((1, H)),
            _rep((H, H)), _rep((H, H)), _rep((H, H)),
        ],
        out_specs=(pl.BlockSpec((TM, 2 * H), lambda i: (i, 0)),
                   pl.BlockSpec((TM, 2 * H), lambda i: (i, 0))),
        compiler_params=cp_arb,
    )(xn, xg,
      ff_node_w1, _r2(ff_node_b1), ff_node_w2, _r2(ff_node_b2),
      ff_graph_w1, _r2(ff_graph_b1), ff_graph_w2, _r2(ff_graph_b2),
      w_ps, w_pdv, w_pdu)

    pp3 = pp.reshape(Np, 1, 2 * H)
    eout, eagg_a, eagg_b = pl.pallas_call(
        functools.partial(_edge_kernel, gE, E),
        out_shape=(jax.ShapeDtypeStruct((Ep, O), F32),
                   jax.ShapeDtypeStruct((Np, 1, 2 * O), F32),
                   jax.ShapeDtypeStruct((Np, 1, 2 * O), F32)),
        grid=(gE + 1,),
        in_specs=[
            pl.BlockSpec((TME, D),
                         lambda i, g=gE: (jnp.minimum(i, g - 1), 0)),
            pl.BlockSpec((1, 1, TME),
                         lambda i, g=gE: (jnp.minimum(i, g - 1), 0, 0),
                         memory_space=pltpu.SMEM),
            pl.BlockSpec((1, 1, TME),
                         lambda i, g=gE: (jnp.minimum(i, g - 1), 0, 0),
                         memory_space=pltpu.SMEM),
            pl.BlockSpec((1, 1, TME),
                         lambda i: (jnp.maximum(i - 1, 0), 0, 0),
                         memory_space=pltpu.SMEM),
            pl.BlockSpec((Np, 1, 2 * H), lambda i: (0, 0, 0)),
            _rep((D, D)), _rep((1, D)), _rep((D, H)), _rep((1, H)),
            _rep((H, H)), _rep((1, H)), _rep((H, O)), _rep((1, O)),
        ],
        out_specs=(pl.BlockSpec((TME, O),
                                lambda i, g=gE: (jnp.minimum(i, g - 1), 0)),
                   pl.BlockSpec((Np, 1, 2 * O), lambda i: (0, 0, 0)),
                   pl.BlockSpec((Np, 1, 2 * O), lambda i: (0, 0, 0))),
        scratch_shapes=[pltpu.VMEM((TME, 2 * H), F32),
                        pltpu.VMEM((TME, 2 * H), F32),
                        pltpu.VMEM((TME, 2 * O), F32)],
        compiler_params=cp_arb,
    )(xe, src_t, dst_t, dst_t, pp3,
      ff_edge_w1, _r2(ff_edge_b1), ff_edge_w2, _r2(ff_edge_b2),
      w_ee, _r2(edge_update_b1), edge_update_w2, _r2(edge_update_b2))

    eagg2a = eagg_a.reshape(Np, 2 * O)
    eagg2b = eagg_b.reshape(Np, 2 * O)
    nout, pool = pl.pallas_call(
        _node_kernel,
        out_shape=(jax.ShapeDtypeStruct((Np, O), F32),
                   jax.ShapeDtypeStruct((Gp, 2 * O + H), F32)),
        grid=(gN,),
        in_specs=[
            pl.BlockSpec((TM, 2 * H), lambda j: (j, 0)),
            pl.BlockSpec((TM, 2 * O), lambda j: (j, 0)),
            pl.BlockSpec((TM, 2 * O), lambda j: (j, 0)),
            pl.BlockSpec((TM, D), lambda j: (j, 0)),
            pl.BlockSpec((1, TM), lambda j: (0, j)),
            _rep((2 * H, H)), _rep((O, H)), _rep((1, H)),
            _rep((H, O)), _rep((1, O)),
        ],
        out_specs=(pl.BlockSpec((TM, O), lambda j: (j, 0)),
                   pl.BlockSpec((Gp, 2 * O + H), lambda j: (0, 0))),
        compiler_params=cp_arb,
    )(vu, eagg2a, eagg2b, xn, ng_r,
      w_vu, w_ef, _r2(node_update_b1), node_update_w2, _r2(node_update_b2))

    gout = pl.pallas_call(
        _graph_kernel,
        out_shape=jax.ShapeDtypeStruct((Np, O), F32),
        grid=(gN,),
        in_specs=[
            _rep((Gp, 2 * O + H)),
            _rep((Gp, 1)), _rep((Gp, 1)),
            pl.BlockSpec((TM, 1), lambda j: (j, 0)),
            pl.BlockSpec((TM, D), lambda j: (j, 0)),
            _rep((2 * O + H, H)), _rep((1, H)), _rep((H, O)), _rep((1, O)),
        ],
        out_specs=pl.BlockSpec((TM, O), lambda j: (j, 0)),
        scratch_shapes=[pltpu.VMEM((Gp, O), F32)],
        compiler_params=cp_arb,
    )(pool, cnt_n, cnt_e, ng_c, xg,
      graph_update_w1, _r2(graph_update_b1),
      graph_update_w2, _r2(graph_update_b2))

    return nout[:N], eout[:E], gout[:N]


# speedup vs baseline: 2.0760x; 1.0489x over previous
import functools

import jax
import jax.numpy as jnp
from jax import lax
from jax.experimental import pallas as pl
from jax.experimental.pallas import tpu as pltpu

NEG_SLOPE = 0.01
TM = 1024
TME = 1024
F32 = jnp.float32
VMEM_LIMIT = 56 * 1024 * 1024


def _lrelu(x):
    return jnp.maximum(x, NEG_SLOPE * x)


def _round_up(n, m):
    return ((n + m - 1) // m) * m


def _pad_rows(x, rows):
    x = x.astype(F32)
    if rows == x.shape[0]:
        return x
    return jnp.pad(x, ((0, rows - x.shape[0]), (0, 0)))


def _pad_idx(idx, rows, fill):
    idx = idx.astype(jnp.int32)
    if rows == idx.shape[0]:
        return idx
    return jnp.pad(idx, (0, rows - idx.shape[0]), constant_values=fill)


def _rep(shape):
    return pl.BlockSpec(shape, lambda *_: (0,) * len(shape))


def _r2(b):
    return b.reshape(1, -1)


def _ff_proj_kernel(xn_ref, xg_ref,
                    wn1, bn1, wn2, bn2,
                    wg1, bg1, wg2, bg2,
                    wps, wpdv, wpdu,
                    vu_ref, pp_ref):
    xn = xn_ref[...]
    hv = _lrelu(jnp.dot(xn, wn1[...], preferred_element_type=F32) + bn1[...])
    v = _lrelu(jnp.dot(hv, wn2[...], preferred_element_type=F32) + bn2[...])
    xg = xg_ref[...]
    hu = _lrelu(jnp.dot(xg, wg1[...], preferred_element_type=F32) + bg1[...])
    u = _lrelu(jnp.dot(hu, wg2[...], preferred_element_type=F32) + bg2[...])
    vu_ref[...] = jnp.concatenate([v, u], axis=1)
    ps = jnp.dot(v, wps[...], preferred_element_type=F32)
    pd = (jnp.dot(v, wpdv[...], preferred_element_type=F32)
          + jnp.dot(u, wpdu[...], preferred_element_type=F32))
    pp_ref[...] = jnp.concatenate([ps, pd], axis=1)


def _edge_kernel(n_tiles, n_edges,
                 xe_ref, src_ref, dst_ref, dstp_ref, pp_ref,
                 we1, be1, we2, be2, wee, b1, w2, b2,
                 eout_ref, ega_ref, egb_ref, gs_scr, gd_scr, pay_scr):
    i = pl.program_id(0)

    @pl.when(i == 0)
    def _():
        nrows = ega_ref.shape[0]
        zblk = jnp.zeros((128,) + ega_ref.shape[1:], ega_ref.dtype)

        def _zero(k, carry):
            ega_ref[pl.ds(k * 128, 128)] = zblk
            egb_ref[pl.ds(k * 128, 128)] = zblk
            return carry

        lax.fori_loop(0, nrows // 128, _zero, 0)
        pay_scr[...] = jnp.zeros(pay_scr.shape, pay_scr.dtype)

    xe = xe_ref[...]
    h = _lrelu(jnp.dot(xe, we1[...], preferred_element_type=F32) + be1[...])
    e = _lrelu(jnp.dot(h, we2[...], preferred_element_type=F32) + be2[...])
    q = jnp.dot(e, wee[...], preferred_element_type=F32) + b1[...]

    CH = 8
    for b in range(TME // CH):
        svals = [pp_ref[src_ref[0, 0, b * CH + j], 0] for j in range(CH)]
        dvals = [pp_ref[dst_ref[0, 0, b * CH + j], 0] for j in range(CH)]
        for j in range(CH):
            gs_scr[b * CH + j] = svals[j]
            gd_scr[b * CH + j] = dvals[j]
        for j in range(CH):
            mi = b * CH + j
            d = dstp_ref[0, 0, mi]
            acc = ega_ref if j % 2 == 0 else egb_ref
            acc[d, 0] = acc[d, 0] + pay_scr[mi]

    hh = gs_scr.shape[1] // 2
    h1 = _lrelu(gs_scr[:, :hh] + gd_scr[:, hh:] + q)
    e_new = _lrelu(jnp.dot(h1, w2[...], preferred_element_type=F32) + b2[...])
    eout_ref[...] = e_new + xe

    base = jnp.minimum(i, n_tiles - 1) * TME
    rows = lax.broadcasted_iota(jnp.int32, (TME, 1), 0) + base
    valid = (rows < n_edges).astype(F32)
    pay_scr[...] = jnp.concatenate([e_new, jnp.ones_like(e_new)], axis=1) * valid


def _node_kernel(vu_ref, ega_ref, egb_ref, xn_ref, ng_ref,
                 w_vu, w_ef, b1, w2, b2,
                 nout_ref, pool_ref):
    j = pl.program_id(0)

    @pl.when(j == 0)
    def _():
        pool_ref[...] = jnp.zeros_like(pool_ref)

    o = nout_ref.shape[1]
    agg = ega_ref[...] + egb_ref[...]
    ef_sum = agg[:, :o]
    deg = agg[:, o:]
    ef = ef_sum * pl.reciprocal(jnp.maximum(deg, 1.0), approx=True)
    vu = vu_ref[...]
    h = _lrelu(jnp.dot(vu, w_vu[...], preferred_element_type=F32)
               + jnp.dot(ef, w_ef[...], preferred_element_type=F32)
               + b1[...])
    n_new = _lrelu(jnp.dot(h, w2[...], preferred_element_type=F32) + b2[...])
    nout_ref[...] = n_new + xn_ref[...]

    gp = pool_ref.shape[0]
    u = vu[:, vu.shape[1] // 2:]
    pooled = jnp.concatenate([n_new, ef_sum, u], axis=1)
    row_ids = lax.broadcasted_iota(jnp.int32, (gp, TM), 0)
    oh = (row_ids == ng_ref[...]).astype(F32)
    pool_ref[...] += jnp.dot(oh, pooled, preferred_element_type=F32)


def _graph_kernel(pool_ref, cntn_ref, cnte_ref, ng_ref, xg_ref,
                  w1, b1, w2, b2,
                  gout_ref, gnew_scr):
    j = pl.program_id(0)
    o = gout_ref.shape[1]

    @pl.when(j == 0)
    def _():
        pool = pool_ref[...]
        gp, width = pool.shape
        inv_n = pl.reciprocal(jnp.maximum(cntn_ref[...], 1.0), approx=True)
        inv_e = pl.reciprocal(jnp.maximum(cnte_ref[...], 1.0), approx=True)
        lane = lax.broadcasted_iota(jnp.int32, (gp, width), 1)
        scale = jnp.where(lane < o, inv_n, jnp.where(lane < 2 * o, inv_e, inv_n))
        cat_g = pool * scale
        hg = _lrelu(jnp.dot(cat_g, w1[...], preferred_element_type=F32) + b1[...])
        gnew_scr[...] = _lrelu(jnp.dot(hg, w2[...], preferred_element_type=F32)
                               + b2[...])

    gp = gnew_scr.shape[0]
    col_ids = lax.broadcasted_iota(jnp.int32, (TM, gp), 1)
    oh = (col_ids == ng_ref[...]).astype(F32)
    gout_ref[...] = (jnp.dot(oh, gnew_scr[...], preferred_element_type=F32)
                     + xg_ref[...])


def kernel(ff_node_w1, ff_node_b1, ff_node_w2, ff_node_b2,
           ff_edge_w1, ff_edge_b1, ff_edge_w2, ff_edge_b2,
           ff_graph_w1, ff_graph_b1, ff_graph_w2, ff_graph_b2,
           edge_update_w1, edge_update_b1, edge_update_w2, edge_update_b2,
           node_update_w1, node_update_b1, node_update_w2, node_update_b2,
           graph_update_w1, graph_update_b1, graph_update_w2, graph_update_b2,
           node_feats, edge_feats, graph_feats,
           src, dst, node_graph, batch_num_nodes, batch_num_edges):
    N, D = node_feats.shape
    E = edge_feats.shape[0]
    G = batch_num_nodes.shape[0]
    H = ff_node_w2.shape[1]
    O = edge_update_w2.shape[1]

    Np = _round_up(max(N, 1), 2 * TM)
    Ep = _round_up(max(E, 1), 2 * TME)
    Gp = _round_up(max(G, 1), 8)
    gN = Np // TM
    gE = Ep // TME

    w_ps = edge_update_w1[0:H]
    w_pdv = edge_update_w1[H:2 * H]
    w_ee = edge_update_w1[2 * H:3 * H]
    w_pdu = edge_update_w1[3 * H:4 * H]
    w_vu = jnp.concatenate([node_update_w1[0:H],
                            node_update_w1[H + O:H + O + H]], axis=0)
    w_ef = node_update_w1[H:H + O]

    xn = _pad_rows(node_feats, Np)
    xg = _pad_rows(graph_feats, Np)
    xe = _pad_rows(edge_feats, Ep)
    src_t = _pad_idx(src, Ep, 0).reshape(Ep // TME, 1, TME)
    dst_t = _pad_idx(dst, Ep, 0).reshape(Ep // TME, 1, TME)
    ng_r = _pad_idx(node_graph, Np, -1).reshape(1, Np)
    ng_c = _pad_idx(node_graph, Np, -1).reshape(Np, 1)
    cnt_n = jnp.zeros((Gp, 1), F32).at[:G, 0].set(batch_num_nodes.astype(F32))
    cnt_e = jnp.zeros((Gp, 1), F32).at[:G, 0].set(batch_num_edges.astype(F32))

    cp_arb = pltpu.CompilerParams(dimension_semantics=("arbitrary",),
                                  vmem_limit_bytes=VMEM_LIMIT)

    vu, pp = pl.pallas_call(
        _ff_proj_kernel,
        out_shape=(jax.ShapeDtypeStruct((Np, 2 * H), F32),
                   jax.ShapeDtypeStruct((Np, 2 * H), F32)),
        grid=(gN,),
        in_specs=[
            pl.BlockSpec((TM, D), lambda i: (i, 0)),
            pl.BlockSpec((TM, D), lambda i: (i, 0)),
            _rep((D, D)), _rep((1, D)), _rep((D, H)), _rep((1, H)),
            _rep((D, D)), _rep((1, D)), _rep((D, H)), _rep((1, H)),
            _rep((H, H)), _rep((H, H)), _rep((H, H)),
        ],
        out_specs=(pl.BlockSpec((TM, 2 * H), lambda i: (i, 0)),
                   pl.BlockSpec((TM, 2 * H), lambda i: (i, 0))),
        compiler_params=cp_arb,
    )(xn, xg,
      ff_node_w1, _r2(ff_node_b1), ff_node_w2, _r2(ff_node_b2),
      ff_graph_w1, _r2(ff_graph_b1), ff_graph_w2, _r2(ff_graph_b2),
      w_ps, w_pdv, w_pdu)

    pp3 = pp.reshape(Np, 1, 2 * H)
    eout, eagg_a, eagg_b = pl.pallas_call(
        functools.partial(_edge_kernel, gE, E),
        out_shape=(jax.ShapeDtypeStruct((Ep, O), F32),
                   jax.ShapeDtypeStruct((Np, 1, 2 * O), F32),
                   jax.ShapeDtypeStruct((Np, 1, 2 * O), F32)),
        grid=(gE + 1,),
        in_specs=[
            pl.BlockSpec((TME, D),
                         lambda i, g=gE: (jnp.minimum(i, g - 1), 0)),
            pl.BlockSpec((1, 1, TME),
                         lambda i, g=gE: (jnp.minimum(i, g - 1), 0, 0),
                         memory_space=pltpu.SMEM),
            pl.BlockSpec((1, 1, TME),
                         lambda i, g=gE: (jnp.minimum(i, g - 1), 0, 0),
                         memory_space=pltpu.SMEM),
            pl.BlockSpec((1, 1, TME),
                         lambda i: (jnp.maximum(i - 1, 0), 0, 0),
                         memory_space=pltpu.SMEM),
            pl.BlockSpec((Np, 1, 2 * H), lambda i: (0, 0, 0)),
            _rep((D, D)), _rep((1, D)), _rep((D, H)), _rep((1, H)),
            _rep((H, H)), _rep((1, H)), _rep((H, O)), _rep((1, O)),
        ],
        out_specs=(pl.BlockSpec((TME, O),
                                lambda i, g=gE: (jnp.minimum(i, g - 1), 0)),
                   pl.BlockSpec((Np, 1, 2 * O), lambda i: (0, 0, 0)),
                   pl.BlockSpec((Np, 1, 2 * O), lambda i: (0, 0, 0))),
        scratch_shapes=[pltpu.VMEM((TME, 2 * H), F32),
                        pltpu.VMEM((TME, 2 * H), F32),
                        pltpu.VMEM((TME, 2 * O), F32)],
        compiler_params=cp_arb,
    )(xe, src_t, dst_t, dst_t, pp3,
      ff_edge_w1, _r2(ff_edge_b1), ff_edge_w2, _r2(ff_edge_b2),
      w_ee, _r2(edge_update_b1), edge_update_w2, _r2(edge_update_b2))

    eagg2a = eagg_a.reshape(Np, 2 * O)
    eagg2b = eagg_b.reshape(Np, 2 * O)
    nout, pool = pl.pallas_call(
        _node_kernel,
        out_shape=(jax.ShapeDtypeStruct((Np, O), F32),
                   jax.ShapeDtypeStruct((Gp, 2 * O + H), F32)),
        grid=(gN,),
        in_specs=[
            pl.BlockSpec((TM, 2 * H), lambda j: (j, 0)),
            pl.BlockSpec((TM, 2 * O), lambda j: (j, 0)),
            pl.BlockSpec((TM, 2 * O), lambda j: (j, 0)),
            pl.BlockSpec((TM, D), lambda j: (j, 0)),
            pl.BlockSpec((1, TM), lambda j: (0, j)),
            _rep((2 * H, H)), _rep((O, H)), _rep((1, H)),
            _rep((H, O)), _rep((1, O)),
        ],
        out_specs=(pl.BlockSpec((TM, O), lambda j: (j, 0)),
                   pl.BlockSpec((Gp, 2 * O + H), lambda j: (0, 0))),
        compiler_params=cp_arb,
    )(vu, eagg2a, eagg2b, xn, ng_r,
      w_vu, w_ef, _r2(node_update_b1), node_update_w2, _r2(node_update_b2))

    gout = pl.pallas_call(
        _graph_kernel,
        out_shape=jax.ShapeDtypeStruct((Np, O), F32),
        grid=(gN,),
        in_specs=[
            _rep((Gp, 2 * O + H)),
            _rep((Gp, 1)), _rep((Gp, 1)),
            pl.BlockSpec((TM, 1), lambda j: (j, 0)),
            pl.BlockSpec((TM, D), lambda j: (j, 0)),
            _rep((2 * O + H, H)), _rep((1, H)), _rep((H, O)), _rep((1, O)),
        ],
        out_specs=pl.BlockSpec((TM, O), lambda j: (j, 0)),
        scratch_shapes=[pltpu.VMEM((Gp, O), F32)],
        compiler_params=cp_arb,
    )(pool, cnt_n, cnt_e, ng_c, xg,
      graph_update_w1, _r2(graph_update_b1),
      graph_update_w2, _r2(graph_update_b2))

    return nout[:N], eout[:E], gout[:N]
